# Initial kernel scaffold; baseline (speedup 1.0000x reference)
import jax
import jax.numpy as jnp
from jax.experimental import pallas as pl


def kernel(x, edge_index, W1, b1, g1, bt1, W2, b2, g2, bt2, We1, be1, We2, be2, Wu1, bu1, Wu2, bu2, ge, bte, Ww, bw):
    raise NotImplementedError("write your pallas kernel here")


# speedup vs baseline: 1.0594x; 1.0594x over previous
import functools

import jax
import jax.numpy as jnp
from jax.experimental import pallas as pl

N = 10000
E = 320000
D = 128
ED = 64
HD = 128

NB = 1000
EB = 2560


def _erf(z):
    return jax.lax.erf(z)


def _gelu(x):
    return 0.5 * x * (1.0 + _erf(x * 0.7071067811865476))


def _ln(x, g, b):
    m = jnp.mean(x, axis=-1, keepdims=True)
    v = jnp.mean((x - m) ** 2, axis=-1, keepdims=True)
    return (x - m) * jax.lax.rsqrt(v + 1e-5) * g + b


def _ka_body(x_ref, w_ref, deg_ref, xcat_ref, dinv_ref):
    xcat_ref[:] = jnp.dot(x_ref[:], w_ref[:], preferred_element_type=jnp.float32)
    dinv_ref[:] = jax.lax.rsqrt(deg_ref[:] + 1.0)


def _node_matmul_dinv(x, wcat, deg):
    ko = wcat.shape[1]
    return pl.pallas_call(
        _ka_body,
        grid=(N // NB,),
        in_specs=[
            pl.BlockSpec((NB, D), lambda i: (i, 0)),
            pl.BlockSpec((D, ko), lambda i: (0, 0)),
            pl.BlockSpec((NB, 1), lambda i: (i, 0)),
        ],
        out_specs=[
            pl.BlockSpec((NB, ko), lambda i: (i, 0)),
            pl.BlockSpec((NB, 1), lambda i: (i, 0)),
        ],
        out_shape=[
            jax.ShapeDtypeStruct((N, ko), jnp.float32),
            jax.ShapeDtypeStruct((N, 1), jnp.float32),
        ],
    )(x, wcat, deg)


def _kb_body(xs_ref, xd_ref, gxw_ref, ds_ref, dd_ref, we2_ref, wu1c_ref,
             be1_ref, be2_ref, bu1_ref, ec_ref, msg_ref):
    e1 = jax.nn.relu(xs_ref[:] + xd_ref[:] + be1_ref[:])
    e = jnp.dot(e1, we2_ref[:], preferred_element_type=jnp.float32) + be2_ref[:]
    ec_ref[:] = jnp.dot(e, wu1c_ref[:], preferred_element_type=jnp.float32) + bu1_ref[:]
    msg_ref[:] = gxw_ref[:] * (ds_ref[:] * dd_ref[:])


def _edge_pass1(xs, xd, gxw, ds, dd, We2, Wu1c, be1, be2, bu1):
    return pl.pallas_call(
        _kb_body,
        grid=(E // EB,),
        in_specs=[
            pl.BlockSpec((EB, ED), lambda i: (i, 0)),
            pl.BlockSpec((EB, ED), lambda i: (i, 0)),
            pl.BlockSpec((EB, D), lambda i: (i, 0)),
            pl.BlockSpec((EB, 1), lambda i: (i, 0)),
            pl.BlockSpec((EB, 1), lambda i: (i, 0)),
            pl.BlockSpec((ED, ED), lambda i: (0, 0)),
            pl.BlockSpec((ED, HD), lambda i: (0, 0)),
            pl.BlockSpec((1, ED), lambda i: (0, 0)),
            pl.BlockSpec((1, ED), lambda i: (0, 0)),
            pl.BlockSpec((1, HD), lambda i: (0, 0)),
        ],
        out_specs=[
            pl.BlockSpec((EB, HD), lambda i: (i, 0)),
            pl.BlockSpec((EB, D), lambda i: (i, 0)),
        ],
        out_shape=[
            jax.ShapeDtypeStruct((E, HD), jnp.float32),
            jax.ShapeDtypeStruct((E, D), jnp.float32),
        ],
    )(xs, xd, gxw, ds, dd, We2, Wu1c,
      be1.reshape(1, ED), be2.reshape(1, ED), bu1.reshape(1, HD))


def _kc_body(acc_ref, xw_ref, dinv_ref, b_ref, g_ref, bt_ref, w_ref, hcat_ref):
    pre = acc_ref[:] + xw_ref[:] * (dinv_ref[:] * dinv_ref[:]) + b_ref[:]
    h = _ln(_gelu(pre), g_ref[:], bt_ref[:])
    hcat_ref[:] = jnp.dot(h, w_ref[:], preferred_element_type=jnp.float32)


def _node_pass2(acc, xw, dinv, b1, g1, bt1, wcat2):
    ko = wcat2.shape[1]
    return pl.pallas_call(
        _kc_body,
        grid=(N // NB,),
        in_specs=[
            pl.BlockSpec((NB, D), lambda i: (i, 0)),
            pl.BlockSpec((NB, D), lambda i: (i, 0)),
            pl.BlockSpec((NB, 1), lambda i: (i, 0)),
            pl.BlockSpec((1, D), lambda i: (0, 0)),
            pl.BlockSpec((1, D), lambda i: (0, 0)),
            pl.BlockSpec((1, D), lambda i: (0, 0)),
            pl.BlockSpec((D, ko), lambda i: (0, 0)),
        ],
        out_specs=pl.BlockSpec((NB, ko), lambda i: (i, 0)),
        out_shape=jax.ShapeDtypeStruct((N, ko), jnp.float32),
    )(acc, xw, dinv, b1.reshape(1, D), g1.reshape(1, D), bt1.reshape(1, D), wcat2)


def _kd_body(has_ref, hbd_ref, ec_ref, wu2_ref, bu2_ref, ge_ref, bte_ref,
             ww_ref, bw_ref, ew_ref):
    u1 = jax.nn.relu(has_ref[:] + hbd_ref[:] + ec_ref[:])
    u = jnp.dot(u1, wu2_ref[:], preferred_element_type=jnp.float32) + bu2_ref[:]
    e2 = _ln(u, ge_ref[:], bte_ref[:])
    logit = jnp.sum(e2 * ww_ref[:], axis=-1, keepdims=True) + bw_ref[:]
    ew_ref[:] = jax.nn.sigmoid(logit)


def _edge_pass2(has, hbd, ec, Wu2, bu2, ge, bte, Ww, bw):
    return pl.pallas_call(
        _kd_body,
        grid=(E // EB,),
        in_specs=[
            pl.BlockSpec((EB, HD), lambda i: (i, 0)),
            pl.BlockSpec((EB, HD), lambda i: (i, 0)),
            pl.BlockSpec((EB, HD), lambda i: (i, 0)),
            pl.BlockSpec((HD, ED), lambda i: (0, 0)),
            pl.BlockSpec((1, ED), lambda i: (0, 0)),
            pl.BlockSpec((1, ED), lambda i: (0, 0)),
            pl.BlockSpec((1, ED), lambda i: (0, 0)),
            pl.BlockSpec((1, ED), lambda i: (0, 0)),
            pl.BlockSpec((1, 1), lambda i: (0, 0)),
        ],
        out_specs=pl.BlockSpec((EB, 1), lambda i: (i, 0)),
        out_shape=jax.ShapeDtypeStruct((E, 1), jnp.float32),
    )(has, hbd, ec, Wu2, bu2.reshape(1, ED), ge.reshape(1, ED),
      bte.reshape(1, ED), Ww.reshape(1, ED), bw.reshape(1, 1))


def _kg_body(g2w_ref, ds_ref, dd_ref, ew_ref, msg_ref):
    msg_ref[:] = g2w_ref[:] * (ds_ref[:] * dd_ref[:] * ew_ref[:])


def _edge_scale2(g2w, ds, dd, ew):
    return pl.pallas_call(
        _kg_body,
        grid=(E // EB,),
        in_specs=[
            pl.BlockSpec((EB, D), lambda i: (i, 0)),
            pl.BlockSpec((EB, 1), lambda i: (i, 0)),
            pl.BlockSpec((EB, 1), lambda i: (i, 0)),
            pl.BlockSpec((EB, 1), lambda i: (i, 0)),
        ],
        out_specs=pl.BlockSpec((EB, D), lambda i: (i, 0)),
        out_shape=jax.ShapeDtypeStruct((E, D), jnp.float32),
    )(g2w, ds, dd, ew)


def _krs_body(deg_ref, o_ref):
    o_ref[:] = jax.lax.rsqrt(deg_ref[:] + 1.0)


def _rsqrt1p(deg):
    return pl.pallas_call(
        _krs_body,
        grid=(N // NB,),
        in_specs=[pl.BlockSpec((NB, 1), lambda i: (i, 0))],
        out_specs=pl.BlockSpec((NB, 1), lambda i: (i, 0)),
        out_shape=jax.ShapeDtypeStruct((N, 1), jnp.float32),
    )(deg)


def _ke_body(acc_ref, hw_ref, dinv_ref, b_ref, g_ref, bt_ref, h2_ref):
    pre = acc_ref[:] + hw_ref[:] * (dinv_ref[:] * dinv_ref[:]) + b_ref[:]
    h2_ref[:] = _ln(_gelu(pre), g_ref[:], bt_ref[:])


def _node_final(acc, hw, dinv, b2, g2, bt2):
    return pl.pallas_call(
        _ke_body,
        grid=(N // NB,),
        in_specs=[
            pl.BlockSpec((NB, D), lambda i: (i, 0)),
            pl.BlockSpec((NB, D), lambda i: (i, 0)),
            pl.BlockSpec((NB, 1), lambda i: (i, 0)),
            pl.BlockSpec((1, D), lambda i: (0, 0)),
            pl.BlockSpec((1, D), lambda i: (0, 0)),
            pl.BlockSpec((1, D), lambda i: (0, 0)),
        ],
        out_specs=pl.BlockSpec((NB, D), lambda i: (i, 0)),
        out_shape=jax.ShapeDtypeStruct((N, D), jnp.float32),
    )(acc, hw, dinv, b2.reshape(1, D), g2.reshape(1, D), bt2.reshape(1, D))


def kernel(x, edge_index, W1, b1, g1, bt1, W2, b2, g2, bt2, We1, be1, We2,
           be2, Wu1, bu1, Wu2, bu2, ge, bte, Ww, bw):
    src = edge_index[0]
    dst = edge_index[1]

    We1a, We1b = We1[:D], We1[D:]
    Wu1a, Wu1b, Wu1c = Wu1[:D], Wu1[D:2 * D], Wu1[2 * D:]

    deg1 = jnp.zeros((N,), jnp.float32).at[dst].add(1.0)

    wcat = jnp.concatenate([W1, We1a, We1b], axis=1)
    xcat, dinv1 = _node_matmul_dinv(x, wcat, deg1.reshape(N, 1))
    xw1 = xcat[:, :D]
    xa = xcat[:, D:D + ED]
    xb = xcat[:, D + ED:]

    d1f = dinv1[:, 0]
    ec, msg1 = _edge_pass1(
        xa[src], xb[dst], xw1[src],
        d1f[src].reshape(E, 1), d1f[dst].reshape(E, 1),
        We2, Wu1c, be1, be2, bu1)

    acc1 = jnp.zeros((N, D), jnp.float32).at[dst].add(msg1)

    wcat2 = jnp.concatenate([W2, Wu1a, Wu1b], axis=1)
    hcat = _node_pass2(acc1, xw1, dinv1, b1, g1, bt1, wcat2)
    h1w2 = hcat[:, :D]
    ha = hcat[:, D:2 * D]
    hb = hcat[:, 2 * D:]

    ew = _edge_pass2(ha[src], hb[dst], ec, Wu2, bu2, ge, bte, Ww, bw)
    ewf = ew[:, 0]

    deg2 = jnp.zeros((N,), jnp.float32).at[dst].add(ewf)
    dinv2 = _rsqrt1p(deg2.reshape(N, 1))
    d2f = dinv2[:, 0]

    msg2 = _edge_scale2(h1w2[src], d2f[src].reshape(E, 1),
                        d2f[dst].reshape(E, 1), ew)
    acc2 = jnp.zeros((N, D), jnp.float32).at[dst].add(msg2)

    h2 = _node_final(acc2, h1w2, dinv2, b2, g2, bt2)
    return h2


# speedup vs baseline: 1.2121x; 1.1442x over previous
import functools

import jax
import jax.numpy as jnp
from jax import lax
from jax.experimental import pallas as pl
from jax.experimental.pallas import tpu as pltpu
from jax.experimental.pallas import tpu_sc as plsc

N = 10000
E = 320000
D = 128
ED = 64
HD = 128

NB = 1000
EB = 2560

_NC = 2
_NS = 16
_NW = _NC * _NS
_EPW = E // _NW
_NPS = N // _NS

_SC_MESH = dict(core_axis_name="c", subcore_axis_name="s")


def _sc_hist(idx_hbm, w_hbm, out_hbm, idxbuf, wbuf, acc):
    cid = lax.axis_index("c")
    sid = lax.axis_index("s")
    wid = sid * _NC + cid
    base = wid * _EPW
    pltpu.sync_copy(idx_hbm.at[pl.ds(base, _EPW)], idxbuf)
    pltpu.sync_copy(w_hbm.at[pl.ds(base, _EPW)], wbuf)
    zeros = jnp.zeros((16,), jnp.float32)

    def _zero(i, _):
        acc[pl.ds(i * 16, 16)] = zeros
        return 0
    lax.fori_loop(0, N // 16, _zero, 0)

    def _accum(j, _):
        s16 = idxbuf[pl.ds(j * 16, 16)]
        w16 = wbuf[pl.ds(j * 16, 16)]
        plsc.addupdate_scatter(acc, [s16], w16)
        return 0
    lax.fori_loop(0, _EPW // 16, _accum, 0)
    pltpu.sync_copy(acc, out_hbm.at[wid])


def _histogram(dst, w):
    k = pl.kernel(
        _sc_hist,
        out_type=jax.ShapeDtypeStruct((_NW, N), jnp.float32),
        mesh=plsc.VectorSubcoreMesh(**_SC_MESH),
        compiler_params=pltpu.CompilerParams(needs_layout_passes=False),
        scratch_types=[
            pltpu.VMEM((_EPW,), jnp.int32),
            pltpu.VMEM((_EPW,), jnp.float32),
            pltpu.VMEM((N,), jnp.float32),
        ],
    )
    return k(dst, w)


_SCHUNK = 200
_NPAD = 10240
_TROWS = _NPAD // _NS
_ZROWS = 128


def _sc_scatter(idx_hbm, msg_hbm, out_hbm, idxbuf, rowsbuf, zbuf, acc_sh):
    cid = lax.axis_index("c")
    sid = lax.axis_index("s")
    wid = sid * _NC + cid
    zeros = jnp.zeros((16,), jnp.float32)

    def _zero(i, _):
        zbuf[i // 8, pl.ds((i % 8) * 16, 16)] = zeros
        return 0
    lax.fori_loop(0, _ZROWS * 8, _zero, 0)
    stripe = sid * _TROWS
    for kk in range(_TROWS // _ZROWS):
        pltpu.sync_copy(zbuf, acc_sh.at[pl.ds(stripe + kk * _ZROWS, _ZROWS)])
    plsc.subcore_barrier()

    def _chunk(ch, _):
        base = wid * _EPW + ch * _SCHUNK
        pltpu.sync_copy(idx_hbm.at[pl.ds(base, _SCHUNK)], idxbuf)
        pltpu.sync_copy(msg_hbm.at[pl.ds(base, _SCHUNK)], rowsbuf)
        pltpu.sync_copy(rowsbuf, acc_sh.at[idxbuf], add=True)
        return 0
    lax.fori_loop(0, _EPW // _SCHUNK, _chunk, 0)
    plsc.subcore_barrier()
    for kk in range(_TROWS // _ZROWS):
        r0 = stripe + kk * _ZROWS
        pltpu.sync_copy(acc_sh.at[pl.ds(r0, _ZROWS)],
                        out_hbm.at[cid, pl.ds(r0, _ZROWS)])


def _scatter_rows(dst, msg):
    k = pl.kernel(
        _sc_scatter,
        out_type=jax.ShapeDtypeStruct((_NC, _NPAD, D), jnp.float32),
        mesh=plsc.VectorSubcoreMesh(**_SC_MESH),
        compiler_params=pltpu.CompilerParams(needs_layout_passes=False),
        scratch_types=[
            pltpu.VMEM((_SCHUNK,), jnp.int32),
            pltpu.VMEM((_SCHUNK, D), jnp.float32),
            pltpu.VMEM((_ZROWS, D), jnp.float32),
            pltpu.VMEM_SHARED((_NPAD, D), jnp.float32),
        ],
    )
    return k(dst, msg)


def _erf(z):
    return jax.lax.erf(z)


def _gelu(x):
    return 0.5 * x * (1.0 + _erf(x * 0.7071067811865476))


def _ln(x, g, b):
    m = jnp.mean(x, axis=-1, keepdims=True)
    v = jnp.mean((x - m) ** 2, axis=-1, keepdims=True)
    return (x - m) * jax.lax.rsqrt(v + 1e-5) * g + b


def _ka_body(x_ref, w_ref, deg_ref, xcat_ref, dinv_ref):
    xcat_ref[:] = jnp.dot(x_ref[:], w_ref[:], preferred_element_type=jnp.float32)
    dinv_ref[:] = jax.lax.rsqrt(deg_ref[:] + 1.0)


def _node_matmul_dinv(x, wcat, deg):
    ko = wcat.shape[1]
    return pl.pallas_call(
        _ka_body,
        grid=(N // NB,),
        in_specs=[
            pl.BlockSpec((NB, D), lambda i: (i, 0)),
            pl.BlockSpec((D, ko), lambda i: (0, 0)),
            pl.BlockSpec((NB, 1), lambda i: (i, 0)),
        ],
        out_specs=[
            pl.BlockSpec((NB, ko), lambda i: (i, 0)),
            pl.BlockSpec((NB, 1), lambda i: (i, 0)),
        ],
        out_shape=[
            jax.ShapeDtypeStruct((N, ko), jnp.float32),
            jax.ShapeDtypeStruct((N, 1), jnp.float32),
        ],
    )(x, wcat, deg)


def _kb_body(xs_ref, xd_ref, gxw_ref, ds_ref, dd_ref, we2_ref, wu1c_ref,
             be1_ref, be2_ref, bu1_ref, ec_ref, msg_ref):
    e1 = jax.nn.relu(xs_ref[:] + xd_ref[:] + be1_ref[:])
    e = jnp.dot(e1, we2_ref[:], preferred_element_type=jnp.float32) + be2_ref[:]
    ec_ref[:] = jnp.dot(e, wu1c_ref[:], preferred_element_type=jnp.float32) + bu1_ref[:]
    msg_ref[:] = gxw_ref[:] * (ds_ref[:] * dd_ref[:])


def _edge_pass1(xs, xd, gxw, ds, dd, We2, Wu1c, be1, be2, bu1):
    return pl.pallas_call(
        _kb_body,
        grid=(E // EB,),
        in_specs=[
            pl.BlockSpec((EB, ED), lambda i: (i, 0)),
            pl.BlockSpec((EB, ED), lambda i: (i, 0)),
            pl.BlockSpec((EB, D), lambda i: (i, 0)),
            pl.BlockSpec((EB, 1), lambda i: (i, 0)),
            pl.BlockSpec((EB, 1), lambda i: (i, 0)),
            pl.BlockSpec((ED, ED), lambda i: (0, 0)),
            pl.BlockSpec((ED, HD), lambda i: (0, 0)),
            pl.BlockSpec((1, ED), lambda i: (0, 0)),
            pl.BlockSpec((1, ED), lambda i: (0, 0)),
            pl.BlockSpec((1, HD), lambda i: (0, 0)),
        ],
        out_specs=[
            pl.BlockSpec((EB, HD), lambda i: (i, 0)),
            pl.BlockSpec((EB, D), lambda i: (i, 0)),
        ],
        out_shape=[
            jax.ShapeDtypeStruct((E, HD), jnp.float32),
            jax.ShapeDtypeStruct((E, D), jnp.float32),
        ],
    )(xs, xd, gxw, ds, dd, We2, Wu1c,
      be1.reshape(1, ED), be2.reshape(1, ED), bu1.reshape(1, HD))


def _kc_body(acc_ref, xw_ref, dinv_ref, b_ref, g_ref, bt_ref, w_ref, hcat_ref):
    pre = (acc_ref[0] + acc_ref[1]
           + xw_ref[:] * (dinv_ref[:] * dinv_ref[:]) + b_ref[:])
    h = _ln(_gelu(pre), g_ref[:], bt_ref[:])
    hcat_ref[:] = jnp.dot(h, w_ref[:], preferred_element_type=jnp.float32)


def _node_pass2(acc, xw, dinv, b1, g1, bt1, wcat2):
    ko = wcat2.shape[1]
    return pl.pallas_call(
        _kc_body,
        grid=(N // NB,),
        in_specs=[
            pl.BlockSpec((_NC, NB, D), lambda i: (0, i, 0)),
            pl.BlockSpec((NB, D), lambda i: (i, 0)),
            pl.BlockSpec((NB, 1), lambda i: (i, 0)),
            pl.BlockSpec((1, D), lambda i: (0, 0)),
            pl.BlockSpec((1, D), lambda i: (0, 0)),
            pl.BlockSpec((1, D), lambda i: (0, 0)),
            pl.BlockSpec((D, ko), lambda i: (0, 0)),
        ],
        out_specs=pl.BlockSpec((NB, ko), lambda i: (i, 0)),
        out_shape=jax.ShapeDtypeStruct((N, ko), jnp.float32),
    )(acc, xw, dinv, b1.reshape(1, D), g1.reshape(1, D), bt1.reshape(1, D), wcat2)


def _kd_body(has_ref, hbd_ref, ec_ref, wu2_ref, bu2_ref, ge_ref, bte_ref,
             ww_ref, bw_ref, ew_ref):
    u1 = jax.nn.relu(has_ref[:] + hbd_ref[:] + ec_ref[:])
    u = jnp.dot(u1, wu2_ref[:], preferred_element_type=jnp.float32) + bu2_ref[:]
    e2 = _ln(u, ge_ref[:], bte_ref[:])
    logit = jnp.sum(e2 * ww_ref[:], axis=-1, keepdims=True) + bw_ref[:]
    ew_ref[:] = jax.nn.sigmoid(logit)


def _edge_pass2(has, hbd, ec, Wu2, bu2, ge, bte, Ww, bw):
    return pl.pallas_call(
        _kd_body,
        grid=(E // EB,),
        in_specs=[
            pl.BlockSpec((EB, HD), lambda i: (i, 0)),
            pl.BlockSpec((EB, HD), lambda i: (i, 0)),
            pl.BlockSpec((EB, HD), lambda i: (i, 0)),
            pl.BlockSpec((HD, ED), lambda i: (0, 0)),
            pl.BlockSpec((1, ED), lambda i: (0, 0)),
            pl.BlockSpec((1, ED), lambda i: (0, 0)),
            pl.BlockSpec((1, ED), lambda i: (0, 0)),
            pl.BlockSpec((1, ED), lambda i: (0, 0)),
            pl.BlockSpec((1, 1), lambda i: (0, 0)),
        ],
        out_specs=pl.BlockSpec((EB, 1), lambda i: (i, 0)),
        out_shape=jax.ShapeDtypeStruct((E, 1), jnp.float32),
    )(has, hbd, ec, Wu2, bu2.reshape(1, ED), ge.reshape(1, ED),
      bte.reshape(1, ED), Ww.reshape(1, ED), bw.reshape(1, 1))


def _kg_body(g2w_ref, ds_ref, dd_ref, ew_ref, msg_ref):
    msg_ref[:] = g2w_ref[:] * (ds_ref[:] * dd_ref[:] * ew_ref[:])


def _edge_scale2(g2w, ds, dd, ew):
    return pl.pallas_call(
        _kg_body,
        grid=(E // EB,),
        in_specs=[
            pl.BlockSpec((EB, D), lambda i: (i, 0)),
            pl.BlockSpec((EB, 1), lambda i: (i, 0)),
            pl.BlockSpec((EB, 1), lambda i: (i, 0)),
            pl.BlockSpec((EB, 1), lambda i: (i, 0)),
        ],
        out_specs=pl.BlockSpec((EB, D), lambda i: (i, 0)),
        out_shape=jax.ShapeDtypeStruct((E, D), jnp.float32),
    )(g2w, ds, dd, ew)


def _krs_body(deg_ref, o_ref):
    o_ref[:] = jax.lax.rsqrt(deg_ref[:] + 1.0)


def _rsqrt1p(deg):
    return pl.pallas_call(
        _krs_body,
        grid=(N // NB,),
        in_specs=[pl.BlockSpec((NB, 1), lambda i: (i, 0))],
        out_specs=pl.BlockSpec((NB, 1), lambda i: (i, 0)),
        out_shape=jax.ShapeDtypeStruct((N, 1), jnp.float32),
    )(deg)


def _ke_body(acc_ref, hw_ref, dinv_ref, b_ref, g_ref, bt_ref, h2_ref):
    pre = (acc_ref[0] + acc_ref[1]
           + hw_ref[:] * (dinv_ref[:] * dinv_ref[:]) + b_ref[:])
    h2_ref[:] = _ln(_gelu(pre), g_ref[:], bt_ref[:])


def _node_final(acc, hw, dinv, b2, g2, bt2):
    return pl.pallas_call(
        _ke_body,
        grid=(N // NB,),
        in_specs=[
            pl.BlockSpec((_NC, NB, D), lambda i: (0, i, 0)),
            pl.BlockSpec((NB, D), lambda i: (i, 0)),
            pl.BlockSpec((NB, 1), lambda i: (i, 0)),
            pl.BlockSpec((1, D), lambda i: (0, 0)),
            pl.BlockSpec((1, D), lambda i: (0, 0)),
            pl.BlockSpec((1, D), lambda i: (0, 0)),
        ],
        out_specs=pl.BlockSpec((NB, D), lambda i: (i, 0)),
        out_shape=jax.ShapeDtypeStruct((N, D), jnp.float32),
    )(acc, hw, dinv, b2.reshape(1, D), g2.reshape(1, D), bt2.reshape(1, D))


def kernel(x, edge_index, W1, b1, g1, bt1, W2, b2, g2, bt2, We1, be1, We2,
           be2, Wu1, bu1, Wu2, bu2, ge, bte, Ww, bw):
    src = edge_index[0]
    dst = edge_index[1]

    We1a, We1b = We1[:D], We1[D:]
    Wu1a, Wu1b, Wu1c = Wu1[:D], Wu1[D:2 * D], Wu1[2 * D:]

    ones_e = jnp.ones((E,), jnp.float32)
    deg1 = _histogram(dst, ones_e).sum(axis=0)

    wcat = jnp.concatenate([W1, We1a, We1b], axis=1)
    xcat, dinv1 = _node_matmul_dinv(x, wcat, deg1.reshape(N, 1))
    xw1 = xcat[:, :D]
    xa = xcat[:, D:D + ED]
    xb = xcat[:, D + ED:]

    d1f = dinv1[:, 0]
    ec, msg1 = _edge_pass1(
        xa[src], xb[dst], xw1[src],
        d1f[src].reshape(E, 1), d1f[dst].reshape(E, 1),
        We2, Wu1c, be1, be2, bu1)

    acc1 = _scatter_rows(dst, msg1)

    wcat2 = jnp.concatenate([W2, Wu1a, Wu1b], axis=1)
    hcat = _node_pass2(acc1, xw1, dinv1, b1, g1, bt1, wcat2)
    h1w2 = hcat[:, :D]
    ha = hcat[:, D:2 * D]
    hb = hcat[:, 2 * D:]

    ew = _edge_pass2(ha[src], hb[dst], ec, Wu2, bu2, ge, bte, Ww, bw)
    ewf = ew[:, 0]

    deg2 = _histogram(dst, ewf).sum(axis=0)
    dinv2 = _rsqrt1p(deg2.reshape(N, 1))
    d2f = dinv2[:, 0]

    msg2 = _edge_scale2(h1w2[src], d2f[src].reshape(E, 1),
                        d2f[dst].reshape(E, 1), ew)
    acc2 = _scatter_rows(dst, msg2)

    h2 = _node_final(acc2, h1w2, dinv2, b2, g2, bt2)
    return h2


# speedup vs baseline: 7.2569x; 5.9870x over previous
import functools

import jax
import jax.numpy as jnp
from jax import lax
from jax.experimental import pallas as pl
from jax.experimental.pallas import tpu as pltpu
from jax.experimental.pallas import tpu_sc as plsc

N = 10000
E = 320000
D = 128
ED = 64
HD = 128

NB = 1000
EB = 2560

_NC = 2
_NS = 16
_NW = _NC * _NS
_EPW = E // _NW
_NPS = N // _NS

_SC_MESH = dict(core_axis_name="c", subcore_axis_name="s")


def _sc_hist(idx_hbm, w_hbm, out_hbm, idxbuf, wbuf, acc):
    cid = lax.axis_index("c")
    sid = lax.axis_index("s")
    wid = sid * _NC + cid
    base = wid * _EPW
    pltpu.sync_copy(idx_hbm.at[pl.ds(base, _EPW)], idxbuf)
    pltpu.sync_copy(w_hbm.at[pl.ds(base, _EPW)], wbuf)
    zeros = jnp.zeros((16,), jnp.float32)

    def _zero(i, _):
        acc[pl.ds(i * 16, 16)] = zeros
        return 0
    lax.fori_loop(0, N // 16, _zero, 0)

    def _accum(j, _):
        s16 = idxbuf[pl.ds(j * 16, 16)]
        w16 = wbuf[pl.ds(j * 16, 16)]
        plsc.addupdate_scatter(acc, [s16], w16)
        return 0
    lax.fori_loop(0, _EPW // 16, _accum, 0)
    pltpu.sync_copy(acc, out_hbm.at[wid])


def _histogram(dst, w):
    k = pl.kernel(
        _sc_hist,
        out_type=jax.ShapeDtypeStruct((_NW, N), jnp.float32),
        mesh=plsc.VectorSubcoreMesh(**_SC_MESH),
        compiler_params=pltpu.CompilerParams(needs_layout_passes=False),
        scratch_types=[
            pltpu.VMEM((_EPW,), jnp.int32),
            pltpu.VMEM((_EPW,), jnp.float32),
            pltpu.VMEM((N,), jnp.float32),
        ],
    )
    return k(dst, w)


_SCHUNK = 200
_NPAD = 10240
_TROWS = _NPAD // _NS
_ZROWS = 128


def _sc_scatter(idx_hbm, msg_hbm, out_hbm, idxbuf, rowsbuf, zbuf, acc_sh):
    cid = lax.axis_index("c")
    sid = lax.axis_index("s")
    wid = sid * _NC + cid
    zeros = jnp.zeros((16,), jnp.float32)

    def _zero(i, _):
        zbuf[i // 8, pl.ds((i % 8) * 16, 16)] = zeros
        return 0
    lax.fori_loop(0, _ZROWS * 8, _zero, 0)
    stripe = sid * _TROWS
    for kk in range(_TROWS // _ZROWS):
        pltpu.sync_copy(zbuf, acc_sh.at[pl.ds(stripe + kk * _ZROWS, _ZROWS)])
    plsc.subcore_barrier()

    def _chunk(ch, _):
        base = wid * _EPW + ch * _SCHUNK
        pltpu.sync_copy(idx_hbm.at[pl.ds(base, _SCHUNK)], idxbuf)
        pltpu.sync_copy(msg_hbm.at[pl.ds(base, _SCHUNK)], rowsbuf)
        pltpu.sync_copy(rowsbuf, acc_sh.at[idxbuf], add=True)
        return 0
    lax.fori_loop(0, _EPW // _SCHUNK, _chunk, 0)
    plsc.subcore_barrier()
    for kk in range(_TROWS // _ZROWS):
        r0 = stripe + kk * _ZROWS
        pltpu.sync_copy(acc_sh.at[pl.ds(r0, _ZROWS)],
                        out_hbm.at[cid, pl.ds(r0, _ZROWS)])


_GCHUNK = 200


def _sc_gather_body(tabA_hbm, tabB_hbm, dinv_hbm, src_hbm, dst_hbm,
                    gA_hbm, gB_hbm, coef_hbm,
                    srcbuf, dstbuf, dinvbuf, coefbuf, rowsA, rowsB,
                    semA, semB):
    cid = lax.axis_index("c")
    sid = lax.axis_index("s")
    wid = sid * _NC + cid
    base = wid * _EPW
    pltpu.sync_copy(src_hbm.at[pl.ds(base, _EPW)], srcbuf)
    pltpu.sync_copy(dst_hbm.at[pl.ds(base, _EPW)], dstbuf)
    pltpu.sync_copy(dinv_hbm, dinvbuf)

    def _coef(j, _):
        s16 = srcbuf[pl.ds(j * 16, 16)]
        d16 = dstbuf[pl.ds(j * 16, 16)]
        v = plsc.load_gather(dinvbuf, [s16]) * plsc.load_gather(dinvbuf, [d16])
        coefbuf[pl.ds(j * 16, 16)] = v
        return 0
    lax.fori_loop(0, _EPW // 16, _coef, 0)
    pltpu.sync_copy(coefbuf, coef_hbm.at[pl.ds(base, _EPW)])

    def _chunk(ch, _):
        off = ch * _GCHUNK
        a = pltpu.async_copy(tabA_hbm.at[srcbuf.at[pl.ds(off, _GCHUNK)]],
                             rowsA, semA)
        b = pltpu.async_copy(tabB_hbm.at[dstbuf.at[pl.ds(off, _GCHUNK)]],
                             rowsB, semB)
        a.wait()
        b.wait()
        pltpu.sync_copy(rowsA, gA_hbm.at[pl.ds(base + off, _GCHUNK)])
        pltpu.sync_copy(rowsB, gB_hbm.at[pl.ds(base + off, _GCHUNK)])
        return 0
    lax.fori_loop(0, _EPW // _GCHUNK, _chunk, 0)


def _gather_pass(tabA, tabB, dinv, src, dst):
    ka = tabA.shape[1]
    kb = tabB.shape[1]
    k = pl.kernel(
        _sc_gather_body,
        out_type=[
            jax.ShapeDtypeStruct((E, ka), jnp.float32),
            jax.ShapeDtypeStruct((E, kb), jnp.float32),
            jax.ShapeDtypeStruct((E,), jnp.float32),
        ],
        mesh=plsc.VectorSubcoreMesh(**_SC_MESH),
        compiler_params=pltpu.CompilerParams(needs_layout_passes=False),
        scratch_types=[
            pltpu.VMEM((_EPW,), jnp.int32),
            pltpu.VMEM((_EPW,), jnp.int32),
            pltpu.VMEM((N,), jnp.float32),
            pltpu.VMEM((_EPW,), jnp.float32),
            pltpu.VMEM((_GCHUNK, ka), jnp.float32),
            pltpu.VMEM((_GCHUNK, kb), jnp.float32),
            pltpu.SemaphoreType.DMA,
            pltpu.SemaphoreType.DMA,
        ],
    )
    return k(tabA, tabB, dinv, src, dst)


def _sc_coef_body(dinv_hbm, src_hbm, dst_hbm, coef_hbm,
                  srcbuf, dstbuf, dinvbuf, coefbuf):
    cid = lax.axis_index("c")
    sid = lax.axis_index("s")
    wid = sid * _NC + cid
    base = wid * _EPW
    pltpu.sync_copy(src_hbm.at[pl.ds(base, _EPW)], srcbuf)
    pltpu.sync_copy(dst_hbm.at[pl.ds(base, _EPW)], dstbuf)
    pltpu.sync_copy(dinv_hbm, dinvbuf)

    def _coef(j, _):
        s16 = srcbuf[pl.ds(j * 16, 16)]
        d16 = dstbuf[pl.ds(j * 16, 16)]
        v = plsc.load_gather(dinvbuf, [s16]) * plsc.load_gather(dinvbuf, [d16])
        coefbuf[pl.ds(j * 16, 16)] = v
        return 0
    lax.fori_loop(0, _EPW // 16, _coef, 0)
    pltpu.sync_copy(coefbuf, coef_hbm.at[pl.ds(base, _EPW)])


def _coef_pass(dinv, src, dst):
    k = pl.kernel(
        _sc_coef_body,
        out_type=jax.ShapeDtypeStruct((E,), jnp.float32),
        mesh=plsc.VectorSubcoreMesh(**_SC_MESH),
        compiler_params=pltpu.CompilerParams(needs_layout_passes=False),
        scratch_types=[
            pltpu.VMEM((_EPW,), jnp.int32),
            pltpu.VMEM((_EPW,), jnp.int32),
            pltpu.VMEM((N,), jnp.float32),
            pltpu.VMEM((_EPW,), jnp.float32),
        ],
    )
    return k(dinv, src, dst)


def _scatter_rows(dst, msg):
    k = pl.kernel(
        _sc_scatter,
        out_type=jax.ShapeDtypeStruct((_NC, _NPAD, D), jnp.float32),
        mesh=plsc.VectorSubcoreMesh(**_SC_MESH),
        compiler_params=pltpu.CompilerParams(needs_layout_passes=False),
        scratch_types=[
            pltpu.VMEM((_SCHUNK,), jnp.int32),
            pltpu.VMEM((_SCHUNK, D), jnp.float32),
            pltpu.VMEM((_ZROWS, D), jnp.float32),
            pltpu.VMEM_SHARED((_NPAD, D), jnp.float32),
        ],
    )
    return k(dst, msg)


def _erf(z):
    return jax.lax.erf(z)


def _gelu(x):
    return 0.5 * x * (1.0 + _erf(x * 0.7071067811865476))


def _ln(x, g, b):
    m = jnp.mean(x, axis=-1, keepdims=True)
    v = jnp.mean((x - m) ** 2, axis=-1, keepdims=True)
    return (x - m) * jax.lax.rsqrt(v + 1e-5) * g + b


def _ka_body(x_ref, w_ref, deg_ref, xcat_ref, dinv_ref):
    xcat_ref[:] = jnp.dot(x_ref[:], w_ref[:], preferred_element_type=jnp.float32)
    dinv_ref[:] = jax.lax.rsqrt(deg_ref[:] + 1.0)


def _node_matmul_dinv(x, wcat, deg):
    ko = wcat.shape[1]
    return pl.pallas_call(
        _ka_body,
        grid=(N // NB,),
        in_specs=[
            pl.BlockSpec((NB, D), lambda i: (i, 0)),
            pl.BlockSpec((D, ko), lambda i: (0, 0)),
            pl.BlockSpec((NB, 1), lambda i: (i, 0)),
        ],
        out_specs=[
            pl.BlockSpec((NB, ko), lambda i: (i, 0)),
            pl.BlockSpec((NB, 1), lambda i: (i, 0)),
        ],
        out_shape=[
            jax.ShapeDtypeStruct((N, ko), jnp.float32),
            jax.ShapeDtypeStruct((N, 1), jnp.float32),
        ],
    )(x, wcat, deg)


def _kb_body(ga_ref, xd_ref, coef_ref, we2_ref, wu1c_ref,
             be1_ref, be2_ref, bu1_ref, ec_ref, msg_ref):
    ga = ga_ref[:]
    e1 = jax.nn.relu(ga[:, D:D + ED] + xd_ref[:, ED:] + be1_ref[:])
    e = jnp.dot(e1, we2_ref[:], preferred_element_type=jnp.float32) + be2_ref[:]
    ec_ref[:] = jnp.dot(e, wu1c_ref[:], preferred_element_type=jnp.float32) + bu1_ref[:]
    msg_ref[:] = ga[:, :D] * coef_ref[:]


def _edge_pass1(gA, gB, coef, We2, Wu1c, be1, be2, bu1):
    return pl.pallas_call(
        _kb_body,
        grid=(E // EB,),
        in_specs=[
            pl.BlockSpec((EB, 2 * D), lambda i: (i, 0)),
            pl.BlockSpec((EB, 2 * ED), lambda i: (i, 0)),
            pl.BlockSpec((EB, 1), lambda i: (i, 0)),
            pl.BlockSpec((ED, ED), lambda i: (0, 0)),
            pl.BlockSpec((ED, HD), lambda i: (0, 0)),
            pl.BlockSpec((1, ED), lambda i: (0, 0)),
            pl.BlockSpec((1, ED), lambda i: (0, 0)),
            pl.BlockSpec((1, HD), lambda i: (0, 0)),
        ],
        out_specs=[
            pl.BlockSpec((EB, HD), lambda i: (i, 0)),
            pl.BlockSpec((EB, D), lambda i: (i, 0)),
        ],
        out_shape=[
            jax.ShapeDtypeStruct((E, HD), jnp.float32),
            jax.ShapeDtypeStruct((E, D), jnp.float32),
        ],
    )(gA, gB, coef, We2, Wu1c,
      be1.reshape(1, ED), be2.reshape(1, ED), bu1.reshape(1, HD))


def _kc_body(acc_ref, xw_ref, dinv_ref, b_ref, g_ref, bt_ref, w_ref, hcat_ref):
    pre = (acc_ref[0] + acc_ref[1]
           + xw_ref[:] * (dinv_ref[:] * dinv_ref[:]) + b_ref[:])
    h = _ln(_gelu(pre), g_ref[:], bt_ref[:])
    hcat_ref[:] = jnp.dot(h, w_ref[:], preferred_element_type=jnp.float32)


def _node_pass2(acc, xw, dinv, b1, g1, bt1, wcat2):
    ko = wcat2.shape[1]
    return pl.pallas_call(
        _kc_body,
        grid=(N // NB,),
        in_specs=[
            pl.BlockSpec((_NC, NB, D), lambda i: (0, i, 0)),
            pl.BlockSpec((NB, D), lambda i: (i, 0)),
            pl.BlockSpec((NB, 1), lambda i: (i, 0)),
            pl.BlockSpec((1, D), lambda i: (0, 0)),
            pl.BlockSpec((1, D), lambda i: (0, 0)),
            pl.BlockSpec((1, D), lambda i: (0, 0)),
            pl.BlockSpec((D, ko), lambda i: (0, 0)),
        ],
        out_specs=pl.BlockSpec((NB, ko), lambda i: (i, 0)),
        out_shape=jax.ShapeDtypeStruct((N, ko), jnp.float32),
    )(acc, xw, dinv, b1.reshape(1, D), g1.reshape(1, D), bt1.reshape(1, D), wcat2)


def _kd_body(has_ref, hbd_ref, ec_ref, wu2_ref, bu2_ref, ge_ref, bte_ref,
             ww_ref, bw_ref, ew_ref):
    u1 = jax.nn.relu(has_ref[:] + hbd_ref[:] + ec_ref[:])
    u = jnp.dot(u1, wu2_ref[:], preferred_element_type=jnp.float32) + bu2_ref[:]
    e2 = _ln(u, ge_ref[:], bte_ref[:])
    logit = jnp.sum(e2 * ww_ref[:], axis=-1, keepdims=True) + bw_ref[:]
    ew_ref[:] = jax.nn.sigmoid(logit)


def _edge_pass2(gC, hbd, ec, Wu2, bu2, ge, bte, Ww, bw):
    return pl.pallas_call(
        _kd_body,
        grid=(E // EB,),
        in_specs=[
            pl.BlockSpec((EB, HD), lambda i: (i, 1)),
            pl.BlockSpec((EB, HD), lambda i: (i, 0)),
            pl.BlockSpec((EB, HD), lambda i: (i, 0)),
            pl.BlockSpec((HD, ED), lambda i: (0, 0)),
            pl.BlockSpec((1, ED), lambda i: (0, 0)),
            pl.BlockSpec((1, ED), lambda i: (0, 0)),
            pl.BlockSpec((1, ED), lambda i: (0, 0)),
            pl.BlockSpec((1, ED), lambda i: (0, 0)),
            pl.BlockSpec((1, 1), lambda i: (0, 0)),
        ],
        out_specs=pl.BlockSpec((EB, 1), lambda i: (i, 0)),
        out_shape=jax.ShapeDtypeStruct((E, 1), jnp.float32),
    )(gC, hbd, ec, Wu2, bu2.reshape(1, ED), ge.reshape(1, ED),
      bte.reshape(1, ED), Ww.reshape(1, ED), bw.reshape(1, 1))


def _kg_body(g2w_ref, c2_ref, ew_ref, msg_ref):
    msg_ref[:] = g2w_ref[:] * (c2_ref[:] * ew_ref[:])


def _edge_scale2(gC, c2, ew):
    return pl.pallas_call(
        _kg_body,
        grid=(E // EB,),
        in_specs=[
            pl.BlockSpec((EB, D), lambda i: (i, 0)),
            pl.BlockSpec((EB, 1), lambda i: (i, 0)),
            pl.BlockSpec((EB, 1), lambda i: (i, 0)),
        ],
        out_specs=pl.BlockSpec((EB, D), lambda i: (i, 0)),
        out_shape=jax.ShapeDtypeStruct((E, D), jnp.float32),
    )(gC, c2, ew)


def _krs_body(deg_ref, o_ref):
    o_ref[:] = jax.lax.rsqrt(deg_ref[:] + 1.0)


def _rsqrt1p(deg):
    return pl.pallas_call(
        _krs_body,
        grid=(N // NB,),
        in_specs=[pl.BlockSpec((NB, 1), lambda i: (i, 0))],
        out_specs=pl.BlockSpec((NB, 1), lambda i: (i, 0)),
        out_shape=jax.ShapeDtypeStruct((N, 1), jnp.float32),
    )(deg)


def _ke_body(acc_ref, hw_ref, dinv_ref, b_ref, g_ref, bt_ref, h2_ref):
    pre = (acc_ref[0] + acc_ref[1]
           + hw_ref[:] * (dinv_ref[:] * dinv_ref[:]) + b_ref[:])
    h2_ref[:] = _ln(_gelu(pre), g_ref[:], bt_ref[:])


def _node_final(acc, hw, dinv, b2, g2, bt2):
    return pl.pallas_call(
        _ke_body,
        grid=(N // NB,),
        in_specs=[
            pl.BlockSpec((_NC, NB, D), lambda i: (0, i, 0)),
            pl.BlockSpec((NB, D), lambda i: (i, 0)),
            pl.BlockSpec((NB, 1), lambda i: (i, 0)),
            pl.BlockSpec((1, D), lambda i: (0, 0)),
            pl.BlockSpec((1, D), lambda i: (0, 0)),
            pl.BlockSpec((1, D), lambda i: (0, 0)),
        ],
        out_specs=pl.BlockSpec((NB, D), lambda i: (i, 0)),
        out_shape=jax.ShapeDtypeStruct((N, D), jnp.float32),
    )(acc, hw, dinv, b2.reshape(1, D), g2.reshape(1, D), bt2.reshape(1, D))


def kernel(x, edge_index, W1, b1, g1, bt1, W2, b2, g2, bt2, We1, be1, We2,
           be2, Wu1, bu1, Wu2, bu2, ge, bte, Ww, bw):
    src = edge_index[0]
    dst = edge_index[1]

    We1a, We1b = We1[:D], We1[D:]
    Wu1a, Wu1b, Wu1c = Wu1[:D], Wu1[D:2 * D], Wu1[2 * D:]

    ones_e = jnp.ones((E,), jnp.float32)
    deg1 = _histogram(dst, ones_e).sum(axis=0)

    wcat = jnp.concatenate([W1, We1a, We1b], axis=1)
    xcat, dinv1 = _node_matmul_dinv(x, wcat, deg1.reshape(N, 1))
    xbt = xcat[:, D:]

    d1f = dinv1[:, 0]
    gA, gB, coef1 = _gather_pass(xcat, xbt, d1f, src, dst)
    ec, msg1 = _edge_pass1(gA, gB, coef1.reshape(E, 1),
                           We2, Wu1c, be1, be2, bu1)

    acc1 = _scatter_rows(dst, msg1)

    wcat2 = jnp.concatenate([W2, Wu1a, Wu1b], axis=1)
    hcat = _node_pass2(acc1, xcat, dinv1, b1, g1, bt1, wcat2)

    gC, gD, _ = _gather_pass(hcat[:, :2 * D], hcat[:, 2 * D:], d1f, src, dst)

    ew = _edge_pass2(gC, gD, ec, Wu2, bu2, ge, bte, Ww, bw)
    ewf = ew[:, 0]

    deg2 = _histogram(dst, ewf).sum(axis=0)
    dinv2 = _rsqrt1p(deg2.reshape(N, 1))
    coef2 = _coef_pass(dinv2[:, 0], src, dst)

    msg2 = _edge_scale2(gC, coef2.reshape(E, 1), ew)
    acc2 = _scatter_rows(dst, msg2)

    h2 = _node_final(acc2, hcat, dinv2, b2, g2, bt2)
    return h2


# speedup vs baseline: 7.4335x; 1.0243x over previous
import functools

import jax
import jax.numpy as jnp
from jax import lax
from jax.experimental import pallas as pl
from jax.experimental.pallas import tpu as pltpu
from jax.experimental.pallas import tpu_sc as plsc

N = 10000
E = 320000
D = 128
ED = 64
HD = 128

NB = 1000
EB = 2560

_NC = 2
_NS = 16
_NW = _NC * _NS
_EPW = E // _NW
_NPS = N // _NS

_SC_MESH = dict(core_axis_name="c", subcore_axis_name="s")


def _sc_hist(idx_hbm, w_hbm, out_hbm, idxbuf, wbuf, acc):
    cid = lax.axis_index("c")
    sid = lax.axis_index("s")
    wid = sid * _NC + cid
    base = wid * _EPW
    pltpu.sync_copy(idx_hbm.at[pl.ds(base, _EPW)], idxbuf)
    pltpu.sync_copy(w_hbm.at[pl.ds(base, _EPW)], wbuf)
    zeros = jnp.zeros((16,), jnp.float32)

    def _zero(i, _):
        acc[pl.ds(i * 16, 16)] = zeros
        return 0
    lax.fori_loop(0, N // 16, _zero, 0)

    def _accum(j, _):
        s16 = idxbuf[pl.ds(j * 16, 16)]
        w16 = wbuf[pl.ds(j * 16, 16)]
        plsc.addupdate_scatter(acc, [s16], w16)
        return 0
    lax.fori_loop(0, _EPW // 16, _accum, 0)
    pltpu.sync_copy(acc, out_hbm.at[wid])


def _histogram(dst, w):
    k = pl.kernel(
        _sc_hist,
        out_type=jax.ShapeDtypeStruct((_NW, N), jnp.float32),
        mesh=plsc.VectorSubcoreMesh(**_SC_MESH),
        compiler_params=pltpu.CompilerParams(needs_layout_passes=False),
        scratch_types=[
            pltpu.VMEM((_EPW,), jnp.int32),
            pltpu.VMEM((_EPW,), jnp.float32),
            pltpu.VMEM((N,), jnp.float32),
        ],
    )
    return k(dst, w)


_SCHUNK = 400
_NPAD = 10240
_NHALF = _NPAD // 2
_ACCR = _NHALF + 8
_EPT = E // _NS
_TROWS2 = _NHALF // _NS
_ZROWS = 32
_RMI = _SCHUNK // 16
_SEG = 4000


def _zero_spmem_half(zbuf, acc_sh, sid):
    zeros = jnp.zeros((16,), jnp.float32)

    def _zero(i, _):
        zbuf[i // 8, pl.ds((i % 8) * 16, 16)] = zeros
        return 0
    lax.fori_loop(0, _ZROWS * 8, _zero, 0)
    stripe = sid * _TROWS2
    for kk in range(_TROWS2 // _ZROWS):
        pltpu.sync_copy(zbuf, acc_sh.at[pl.ds(stripe + kk * _ZROWS, _ZROWS)])

    @pl.when(sid == 0)
    def _():
        pltpu.sync_copy(zbuf.at[pl.ds(0, 8)], acc_sh.at[pl.ds(_NHALF, 8)])


def _remap_chunk(dstbuf, off, lo, ibuf):
    def _rm(j, _):
        d16 = dstbuf[pl.ds(off + j * 16, 16)]
        v = d16 - lo
        oob = (v < 0) | (v >= _NHALF)
        ibuf[pl.ds(j * 16, 16)] = jnp.where(oob, _NHALF, v)
        return 0
    lax.fori_loop(0, _RMI, _rm, 0)


def _copyout_half(acc_sh, out_hbm, cid, sid):
    stripe = sid * _TROWS2
    for kk in range(_TROWS2 // _ZROWS):
        r0 = stripe + kk * _ZROWS
        pltpu.sync_copy(acc_sh.at[pl.ds(r0, _ZROWS)],
                        out_hbm.at[pl.ds(cid * _NHALF + r0, _ZROWS)])


def _sc_scatter(dst_hbm, msg_hbm, out_hbm,
                dstbuf, idx, rows, zbuf, acc_sh, sem):
    cid = lax.axis_index("c")
    sid = lax.axis_index("s")
    _zero_spmem_half(zbuf, acc_sh, sid)
    base = sid * _EPT
    pltpu.sync_copy(dst_hbm.at[pl.ds(base, _EPT)], dstbuf)
    lo = cid * _NHALF
    plsc.subcore_barrier()

    def _chunk(ch, _):
        off = ch * _SCHUNK
        d = pltpu.async_copy(msg_hbm.at[pl.ds(base + off, _SCHUNK)],
                             rows, sem)
        _remap_chunk(dstbuf, off, lo, idx)
        d.wait()
        pltpu.sync_copy(rows, acc_sh.at[idx], add=True)
        return 0
    lax.fori_loop(0, _EPT // _SCHUNK, _chunk, 0)
    plsc.subcore_barrier()
    _copyout_half(acc_sh, out_hbm, cid, sid)


_GCHUNK = 200


def _sc_gather_body(tabA_hbm, tabB_hbm, src_hbm, dst_hbm,
                    gA_hbm, gB_hbm,
                    srcbuf, dstbuf, rowsA, rowsB, semA, semB):
    cid = lax.axis_index("c")
    sid = lax.axis_index("s")
    wid = sid * _NC + cid
    base = wid * _EPW
    pltpu.sync_copy(src_hbm.at[pl.ds(base, _EPW)], srcbuf)
    pltpu.sync_copy(dst_hbm.at[pl.ds(base, _EPW)], dstbuf)

    def _chunk(ch, _):
        off = ch * _GCHUNK
        a = pltpu.async_copy(tabA_hbm.at[srcbuf.at[pl.ds(off, _GCHUNK)]],
                             rowsA, semA)
        b = pltpu.async_copy(tabB_hbm.at[dstbuf.at[pl.ds(off, _GCHUNK)]],
                             rowsB, semB)
        a.wait()
        b.wait()
        pltpu.sync_copy(rowsA, gA_hbm.at[pl.ds(base + off, _GCHUNK)])
        pltpu.sync_copy(rowsB, gB_hbm.at[pl.ds(base + off, _GCHUNK)])
        return 0
    lax.fori_loop(0, _EPW // _GCHUNK, _chunk, 0)


def _gather_pass(tabA, tabB, src, dst):
    ka = tabA.shape[1]
    kb = tabB.shape[1]
    k = pl.kernel(
        _sc_gather_body,
        out_type=[
            jax.ShapeDtypeStruct((E, ka), jnp.float32),
            jax.ShapeDtypeStruct((E, kb), jnp.float32),
        ],
        mesh=plsc.VectorSubcoreMesh(**_SC_MESH),
        compiler_params=pltpu.CompilerParams(needs_layout_passes=False),
        scratch_types=[
            pltpu.VMEM((_EPW,), jnp.int32),
            pltpu.VMEM((_EPW,), jnp.int32),
            pltpu.VMEM((_GCHUNK, ka), jnp.float32),
            pltpu.VMEM((_GCHUNK, kb), jnp.float32),
            pltpu.SemaphoreType.DMA,
            pltpu.SemaphoreType.DMA,
        ],
    )
    return k(tabA, tabB, src, dst)


def _sc_gather1_body(tab_hbm, src_hbm, g_hbm, srcbuf, rows, sem):
    cid = lax.axis_index("c")
    sid = lax.axis_index("s")
    wid = sid * _NC + cid
    base = wid * _EPW
    pltpu.sync_copy(src_hbm.at[pl.ds(base, _EPW)], srcbuf)

    def _chunk(ch, _):
        off = ch * _GCHUNK
        pltpu.async_copy(tab_hbm.at[srcbuf.at[pl.ds(off, _GCHUNK)]],
                         rows, sem).wait()
        pltpu.sync_copy(rows, g_hbm.at[pl.ds(base + off, _GCHUNK)])
        return 0
    lax.fori_loop(0, _EPW // _GCHUNK, _chunk, 0)


def _gather_one(tab, src):
    ka = tab.shape[1]
    k = pl.kernel(
        _sc_gather1_body,
        out_type=jax.ShapeDtypeStruct((E, ka), jnp.float32),
        mesh=plsc.VectorSubcoreMesh(**_SC_MESH),
        compiler_params=pltpu.CompilerParams(needs_layout_passes=False),
        scratch_types=[
            pltpu.VMEM((_EPW,), jnp.int32),
            pltpu.VMEM((_GCHUNK, ka), jnp.float32),
            pltpu.SemaphoreType.DMA,
        ],
    )
    return k(tab, src)


def _sc_coef_body(dinv_hbm, src_hbm, ew_hbm, coef_hbm,
                  srcbuf, ewbuf, dinvbuf, coefbuf):
    cid = lax.axis_index("c")
    sid = lax.axis_index("s")
    wid = sid * _NC + cid
    base = wid * _EPW
    pltpu.sync_copy(src_hbm.at[pl.ds(base, _EPW)], srcbuf)
    pltpu.sync_copy(ew_hbm.at[pl.ds(base, _EPW)], ewbuf)
    pltpu.sync_copy(dinv_hbm, dinvbuf)

    def _coef(j, _):
        s16 = srcbuf[pl.ds(j * 16, 16)]
        w16 = ewbuf[pl.ds(j * 16, 16)]
        coefbuf[pl.ds(j * 16, 16)] = plsc.load_gather(dinvbuf, [s16]) * w16
        return 0
    lax.fori_loop(0, _EPW // 16, _coef, 0)
    pltpu.sync_copy(coefbuf, coef_hbm.at[pl.ds(base, _EPW)])


def _coef_pass(dinv, src, ew):
    k = pl.kernel(
        _sc_coef_body,
        out_type=jax.ShapeDtypeStruct((E,), jnp.float32),
        mesh=plsc.VectorSubcoreMesh(**_SC_MESH),
        compiler_params=pltpu.CompilerParams(needs_layout_passes=False),
        scratch_types=[
            pltpu.VMEM((_EPW,), jnp.int32),
            pltpu.VMEM((_EPW,), jnp.float32),
            pltpu.VMEM((N,), jnp.float32),
            pltpu.VMEM((_EPW,), jnp.float32),
        ],
    )
    return k(dinv, src, ew)


def _scatter_rows(dst, msg):
    k = pl.kernel(
        _sc_scatter,
        out_type=jax.ShapeDtypeStruct((_NPAD, D), jnp.float32),
        mesh=plsc.VectorSubcoreMesh(**_SC_MESH),
        compiler_params=pltpu.CompilerParams(needs_layout_passes=False),
        scratch_types=[
            pltpu.VMEM((_EPT,), jnp.int32),
            pltpu.VMEM((_SCHUNK,), jnp.int32),
            pltpu.VMEM((_SCHUNK, D), jnp.float32),
            pltpu.VMEM((_ZROWS, D), jnp.float32),
            pltpu.VMEM_SHARED((_ACCR, D), jnp.float32),
            pltpu.SemaphoreType.DMA,
        ],
    )
    return k(dst, msg)


def _erf(z):
    return jax.lax.erf(z)


def _gelu(x):
    return 0.5 * x * (1.0 + _erf(x * 0.7071067811865476))


def _ln(x, g, b):
    m = jnp.mean(x, axis=-1, keepdims=True)
    v = jnp.mean((x - m) ** 2, axis=-1, keepdims=True)
    return (x - m) * jax.lax.rsqrt(v + 1e-5) * g + b


def _ka_body(x_ref, w_ref, deg_ref, tab_ref, xw1s_ref, dinv_ref):
    xc = jnp.dot(x_ref[:], w_ref[:], preferred_element_type=jnp.float32)
    dinv = jax.lax.rsqrt(deg_ref[:] + 1.0)
    tab_ref[:] = xc[:, D:]
    xw1s_ref[:] = xc[:, :D] * dinv
    dinv_ref[:] = dinv


def _node_matmul_dinv(x, wcat, deg):
    ko = wcat.shape[1]
    return pl.pallas_call(
        _ka_body,
        grid=(N // NB,),
        in_specs=[
            pl.BlockSpec((NB, D), lambda i: (i, 0)),
            pl.BlockSpec((D, ko), lambda i: (0, 0)),
            pl.BlockSpec((NB, 1), lambda i: (i, 0)),
        ],
        out_specs=[
            pl.BlockSpec((NB, ko - D), lambda i: (i, 0)),
            pl.BlockSpec((NB, D), lambda i: (i, 0)),
            pl.BlockSpec((NB, 1), lambda i: (i, 0)),
        ],
        out_shape=[
            jax.ShapeDtypeStruct((N, ko - D), jnp.float32),
            jax.ShapeDtypeStruct((N, D), jnp.float32),
            jax.ShapeDtypeStruct((N, 1), jnp.float32),
        ],
    )(x, wcat, deg)


def _kb_body(xs_ref, xd_ref, we2_ref, wu1c_ref,
             be1_ref, be2_ref, bu1_ref, ec_ref):
    e1 = jax.nn.relu(xs_ref[:, :ED] + xd_ref[:, ED:] + be1_ref[:])
    e = jnp.dot(e1, we2_ref[:], preferred_element_type=jnp.float32) + be2_ref[:]
    ec_ref[:] = jnp.dot(e, wu1c_ref[:], preferred_element_type=jnp.float32) + bu1_ref[:]


def _edge_pass1(gS, gD1, We2, Wu1c, be1, be2, bu1):
    return pl.pallas_call(
        _kb_body,
        grid=(E // EB,),
        in_specs=[
            pl.BlockSpec((EB, 2 * ED), lambda i: (i, 0)),
            pl.BlockSpec((EB, 2 * ED), lambda i: (i, 0)),
            pl.BlockSpec((ED, ED), lambda i: (0, 0)),
            pl.BlockSpec((ED, HD), lambda i: (0, 0)),
            pl.BlockSpec((1, ED), lambda i: (0, 0)),
            pl.BlockSpec((1, ED), lambda i: (0, 0)),
            pl.BlockSpec((1, HD), lambda i: (0, 0)),
        ],
        out_specs=pl.BlockSpec((EB, HD), lambda i: (i, 0)),
        out_shape=jax.ShapeDtypeStruct((E, HD), jnp.float32),
    )(gS, gD1, We2, Wu1c,
      be1.reshape(1, ED), be2.reshape(1, ED), bu1.reshape(1, HD))


def _kc_body(acc_ref, xw_ref, dinv_ref, b_ref, g_ref, bt_ref, w_ref,
             tabc_ref, tabd_ref):
    dinv = dinv_ref[:]
    pre = dinv * (acc_ref[:] + xw_ref[:]) + b_ref[:]
    h = _ln(_gelu(pre), g_ref[:], bt_ref[:])
    hcat = jnp.dot(h, w_ref[:], preferred_element_type=jnp.float32)
    tabc_ref[:] = hcat[:, :2 * D]
    tabd_ref[:] = hcat[:, 2 * D:]


def _node_pass2(acc, xw1s, dinv, b1, g1, bt1, wcat2):
    return pl.pallas_call(
        _kc_body,
        grid=(N // NB,),
        in_specs=[
            pl.BlockSpec((NB, D), lambda i: (i, 0)),
            pl.BlockSpec((NB, D), lambda i: (i, 0)),
            pl.BlockSpec((NB, 1), lambda i: (i, 0)),
            pl.BlockSpec((1, D), lambda i: (0, 0)),
            pl.BlockSpec((1, D), lambda i: (0, 0)),
            pl.BlockSpec((1, D), lambda i: (0, 0)),
            pl.BlockSpec((D, 3 * D), lambda i: (0, 0)),
        ],
        out_specs=[
            pl.BlockSpec((NB, 2 * D), lambda i: (i, 0)),
            pl.BlockSpec((NB, D), lambda i: (i, 0)),
        ],
        out_shape=[
            jax.ShapeDtypeStruct((N, 2 * D), jnp.float32),
            jax.ShapeDtypeStruct((N, D), jnp.float32),
        ],
    )(acc, xw1s, dinv, b1.reshape(1, D), g1.reshape(1, D), bt1.reshape(1, D),
      wcat2)


def _kd_body(has_ref, hbd_ref, ec_ref, wu2_ref, bu2_ref, ge_ref, bte_ref,
             ww_ref, bw_ref, ew_ref):
    u1 = jax.nn.relu(has_ref[:] + hbd_ref[:] + ec_ref[:])
    u = jnp.dot(u1, wu2_ref[:], preferred_element_type=jnp.float32) + bu2_ref[:]
    e2 = _ln(u, ge_ref[:], bte_ref[:])
    logit = jnp.sum(e2 * ww_ref[:], axis=-1, keepdims=True) + bw_ref[:]
    ew_ref[:] = jax.nn.sigmoid(logit)


def _edge_pass2(gC, hbd, ec, Wu2, bu2, ge, bte, Ww, bw):
    return pl.pallas_call(
        _kd_body,
        grid=(E // EB,),
        in_specs=[
            pl.BlockSpec((EB, HD), lambda i: (i, 1)),
            pl.BlockSpec((EB, HD), lambda i: (i, 0)),
            pl.BlockSpec((EB, HD), lambda i: (i, 0)),
            pl.BlockSpec((HD, ED), lambda i: (0, 0)),
            pl.BlockSpec((1, ED), lambda i: (0, 0)),
            pl.BlockSpec((1, ED), lambda i: (0, 0)),
            pl.BlockSpec((1, ED), lambda i: (0, 0)),
            pl.BlockSpec((1, ED), lambda i: (0, 0)),
            pl.BlockSpec((1, 1), lambda i: (0, 0)),
        ],
        out_specs=pl.BlockSpec((EB, 1), lambda i: (i, 0)),
        out_shape=jax.ShapeDtypeStruct((E, 1), jnp.float32),
    )(gC, hbd, ec, Wu2, bu2.reshape(1, ED), ge.reshape(1, ED),
      bte.reshape(1, ED), Ww.reshape(1, ED), bw.reshape(1, 1))


def _kg_body(g2w_ref, m2_ref, msg_ref):
    msg_ref[:] = g2w_ref[:] * m2_ref[:]


def _edge_scale2(gC, m2):
    return pl.pallas_call(
        _kg_body,
        grid=(E // EB,),
        in_specs=[
            pl.BlockSpec((EB, D), lambda i: (i, 0)),
            pl.BlockSpec((EB, 1), lambda i: (i, 0)),
        ],
        out_specs=pl.BlockSpec((EB, D), lambda i: (i, 0)),
        out_shape=jax.ShapeDtypeStruct((E, D), jnp.float32),
    )(gC, m2)


def _kf_body(deg_ref, hw_ref, dinv_ref, hws_ref):
    dinv = jax.lax.rsqrt(deg_ref[:] + 1.0)
    dinv_ref[:] = dinv
    hws_ref[:] = hw_ref[:] * dinv


def _dinv2_scale(deg, tabC):
    return pl.pallas_call(
        _kf_body,
        grid=(N // NB,),
        in_specs=[
            pl.BlockSpec((NB, 1), lambda i: (i, 0)),
            pl.BlockSpec((NB, D), lambda i: (i, 0)),
        ],
        out_specs=[
            pl.BlockSpec((NB, 1), lambda i: (i, 0)),
            pl.BlockSpec((NB, D), lambda i: (i, 0)),
        ],
        out_shape=[
            jax.ShapeDtypeStruct((N, 1), jnp.float32),
            jax.ShapeDtypeStruct((N, D), jnp.float32),
        ],
    )(deg, tabC)


def _ke_body(acc_ref, hws_ref, dinv_ref, b_ref, g_ref, bt_ref, h2_ref):
    pre = dinv_ref[:] * (acc_ref[:] + hws_ref[:]) + b_ref[:]
    h2_ref[:] = _ln(_gelu(pre), g_ref[:], bt_ref[:])


def _node_final(acc, hws, dinv, b2, g2, bt2):
    return pl.pallas_call(
        _ke_body,
        grid=(N // NB,),
        in_specs=[
            pl.BlockSpec((NB, D), lambda i: (i, 0)),
            pl.BlockSpec((NB, D), lambda i: (i, 0)),
            pl.BlockSpec((NB, 1), lambda i: (i, 0)),
            pl.BlockSpec((1, D), lambda i: (0, 0)),
            pl.BlockSpec((1, D), lambda i: (0, 0)),
            pl.BlockSpec((1, D), lambda i: (0, 0)),
        ],
        out_specs=pl.BlockSpec((NB, D), lambda i: (i, 0)),
        out_shape=jax.ShapeDtypeStruct((N, D), jnp.float32),
    )(acc, hws, dinv, b2.reshape(1, D), g2.reshape(1, D), bt2.reshape(1, D))


def kernel(x, edge_index, W1, b1, g1, bt1, W2, b2, g2, bt2, We1, be1, We2,
           be2, Wu1, bu1, Wu2, bu2, ge, bte, Ww, bw):
    src = edge_index[0]
    dst = edge_index[1]

    We1a, We1b = We1[:D], We1[D:]
    Wu1a, Wu1b, Wu1c = Wu1[:D], Wu1[D:2 * D], Wu1[2 * D:]

    ones_e = jnp.ones((E,), jnp.float32)
    deg1 = _histogram(dst, ones_e).sum(axis=0)

    wcat = jnp.concatenate([W1, We1a, We1b], axis=1)
    tabAB, xw1s, dinv1 = _node_matmul_dinv(x, wcat, deg1.reshape(N, 1))

    gX = _gather_one(xw1s, src)
    acc1 = _scatter_rows(dst, gX)

    gS, gD1 = _gather_pass(tabAB, tabAB, src, dst)
    ec = _edge_pass1(gS, gD1, We2, Wu1c, be1, be2, bu1)

    wcat2 = jnp.concatenate([W2, Wu1a, Wu1b], axis=1)
    tabC, tabD = _node_pass2(acc1, xw1s, dinv1, b1, g1, bt1, wcat2)

    gC, gDD = _gather_pass(tabC, tabD, src, dst)

    ew = _edge_pass2(gC, gDD, ec, Wu2, bu2, ge, bte, Ww, bw)
    ewf = ew[:, 0]

    deg2 = _histogram(dst, ewf).sum(axis=0)
    dinv2, h1w2s = _dinv2_scale(deg2.reshape(N, 1), tabC)
    m2 = _coef_pass(dinv2[:, 0], src, ewf)

    msg2 = _edge_scale2(gC, m2.reshape(E, 1))
    acc2 = _scatter_rows(dst, msg2)

    h2 = _node_final(acc2, h1w2s, dinv2, b2, g2, bt2)
    return h2


# speedup vs baseline: 8.5211x; 1.1463x over previous
import functools

import jax
import jax.numpy as jnp
from jax import lax
from jax.experimental import pallas as pl
from jax.experimental.pallas import tpu as pltpu
from jax.experimental.pallas import tpu_sc as plsc

N = 10000
E = 320000
D = 128
ED = 64
HD = 128

NB = 1000
EB = 2560

_NC = 2
_NS = 16
_NW = _NC * _NS
_EPW = E // _NW
_NPS = N // _NS

_SC_MESH = dict(core_axis_name="c", subcore_axis_name="s")


def _sc_hist(idx_hbm, w_hbm, out_hbm, idxbuf, wbuf, acc):
    cid = lax.axis_index("c")
    sid = lax.axis_index("s")
    wid = sid * _NC + cid
    base = wid * _EPW
    pltpu.sync_copy(idx_hbm.at[pl.ds(base, _EPW)], idxbuf)
    pltpu.sync_copy(w_hbm.at[pl.ds(base, _EPW)], wbuf)
    zeros = jnp.zeros((16,), jnp.float32)

    def _zero(i, _):
        acc[pl.ds(i * 16, 16)] = zeros
        return 0
    lax.fori_loop(0, N // 16, _zero, 0)

    def _accum(j, _):
        s16 = idxbuf[pl.ds(j * 16, 16)]
        w16 = wbuf[pl.ds(j * 16, 16)]
        plsc.addupdate_scatter(acc, [s16], w16)
        return 0
    lax.fori_loop(0, _EPW // 16, _accum, 0)
    pltpu.sync_copy(acc, out_hbm.at[wid])


def _histogram(dst, w):
    k = pl.kernel(
        _sc_hist,
        out_type=jax.ShapeDtypeStruct((_NW, N), jnp.float32),
        mesh=plsc.VectorSubcoreMesh(**_SC_MESH),
        compiler_params=pltpu.CompilerParams(needs_layout_passes=False),
        scratch_types=[
            pltpu.VMEM((_EPW,), jnp.int32),
            pltpu.VMEM((_EPW,), jnp.float32),
            pltpu.VMEM((N,), jnp.float32),
        ],
    )
    return k(dst, w)


_SCHUNK = 200
_NPAD = 10240
_NHALF = _NPAD // 2
_ACCR = _NHALF + 8
_EPT = E // _NS
_TROWS2 = _NHALF // _NS
_TROWS = _NPAD // _NS
_ZROWS = 32
_RMI = _SCHUNK // 16
_SEG = 4000


def _zero_spmem_half(zbuf, acc_sh, sid):
    zeros = jnp.zeros((16,), jnp.float32)

    def _zero(i, _):
        zbuf[i // 8, pl.ds((i % 8) * 16, 16)] = zeros
        return 0
    lax.fori_loop(0, _ZROWS * 8, _zero, 0)
    stripe = sid * _TROWS2
    for kk in range(_TROWS2 // _ZROWS):
        pltpu.sync_copy(zbuf, acc_sh.at[pl.ds(stripe + kk * _ZROWS, _ZROWS)])

    @pl.when(sid == 0)
    def _():
        pltpu.sync_copy(zbuf.at[pl.ds(0, 8)], acc_sh.at[pl.ds(_NHALF, 8)])


def _remap_chunk(dstbuf, off, lo, ibuf):
    def _rm(j, _):
        d16 = dstbuf[pl.ds(off + j * 16, 16)]
        v = d16 - lo
        oob = (v < 0) | (v >= _NHALF)
        ibuf[pl.ds(j * 16, 16)] = jnp.where(oob, _NHALF, v)
        return 0
    lax.fori_loop(0, _RMI, _rm, 0)


def _copyout_half(acc_sh, out_hbm, cid, sid):
    stripe = sid * _TROWS2
    for kk in range(_TROWS2 // _ZROWS):
        r0 = stripe + kk * _ZROWS
        pltpu.sync_copy(acc_sh.at[pl.ds(r0, _ZROWS)],
                        out_hbm.at[pl.ds(cid * _NHALF + r0, _ZROWS)])


def _sc_scatter(dst_hbm, msg_hbm, out_hbm, idxbuf, rows, zbuf, acc_sh, sem):
    cid = lax.axis_index("c")
    sid = lax.axis_index("s")
    wid = sid * _NC + cid
    zeros = jnp.zeros((16,), jnp.float32)

    def _zero(i, _):
        zbuf[i // 8, pl.ds((i % 8) * 16, 16)] = zeros
        return 0
    lax.fori_loop(0, _ZROWS * 8, _zero, 0)
    stripe = sid * _TROWS
    for kk in range(_TROWS // _ZROWS):
        pltpu.sync_copy(zbuf, acc_sh.at[pl.ds(stripe + kk * _ZROWS, _ZROWS)])
    plsc.subcore_barrier()

    def _chunk(ch, _):
        base = wid * _EPW + ch * _SCHUNK
        d = pltpu.async_copy(msg_hbm.at[pl.ds(base, _SCHUNK)], rows, sem)
        pltpu.sync_copy(dst_hbm.at[pl.ds(base, _SCHUNK)], idxbuf)
        d.wait()
        pltpu.sync_copy(rows, acc_sh.at[idxbuf], add=True)
        return 0
    lax.fori_loop(0, _EPW // _SCHUNK, _chunk, 0)
    plsc.subcore_barrier()
    for kk in range(_TROWS // _ZROWS):
        r0 = stripe + kk * _ZROWS
        pltpu.sync_copy(acc_sh.at[pl.ds(r0, _ZROWS)],
                        out_hbm.at[cid, pl.ds(r0, _ZROWS)])


_GCHUNK = 200


def _sc_gather_body(tabA_hbm, tabB_hbm, src_hbm, dst_hbm,
                    gA_hbm, gB_hbm,
                    srcbuf, dstbuf, rowsA, rowsB, semA, semB):
    cid = lax.axis_index("c")
    sid = lax.axis_index("s")
    wid = sid * _NC + cid
    base = wid * _EPW
    pltpu.sync_copy(src_hbm.at[pl.ds(base, _EPW)], srcbuf)
    pltpu.sync_copy(dst_hbm.at[pl.ds(base, _EPW)], dstbuf)

    def _chunk(ch, _):
        off = ch * _GCHUNK
        a = pltpu.async_copy(tabA_hbm.at[srcbuf.at[pl.ds(off, _GCHUNK)]],
                             rowsA, semA)
        b = pltpu.async_copy(tabB_hbm.at[dstbuf.at[pl.ds(off, _GCHUNK)]],
                             rowsB, semB)
        a.wait()
        b.wait()
        pltpu.sync_copy(rowsA, gA_hbm.at[pl.ds(base + off, _GCHUNK)])
        pltpu.sync_copy(rowsB, gB_hbm.at[pl.ds(base + off, _GCHUNK)])
        return 0
    lax.fori_loop(0, _EPW // _GCHUNK, _chunk, 0)


def _gather_pass(tabA, tabB, src, dst):
    ka = tabA.shape[1]
    kb = tabB.shape[1]
    k = pl.kernel(
        _sc_gather_body,
        out_type=[
            jax.ShapeDtypeStruct((E, ka), jnp.float32),
            jax.ShapeDtypeStruct((E, kb), jnp.float32),
        ],
        mesh=plsc.VectorSubcoreMesh(**_SC_MESH),
        compiler_params=pltpu.CompilerParams(needs_layout_passes=False),
        scratch_types=[
            pltpu.VMEM((_EPW,), jnp.int32),
            pltpu.VMEM((_EPW,), jnp.int32),
            pltpu.VMEM((_GCHUNK, ka), jnp.float32),
            pltpu.VMEM((_GCHUNK, kb), jnp.float32),
            pltpu.SemaphoreType.DMA,
            pltpu.SemaphoreType.DMA,
        ],
    )
    return k(tabA, tabB, src, dst)


def _sc_gather1_body(tab_hbm, src_hbm, g_hbm, srcbuf, rows, sem):
    cid = lax.axis_index("c")
    sid = lax.axis_index("s")
    wid = sid * _NC + cid
    base = wid * _EPW
    pltpu.sync_copy(src_hbm.at[pl.ds(base, _EPW)], srcbuf)

    def _chunk(ch, _):
        off = ch * _GCHUNK
        pltpu.async_copy(tab_hbm.at[srcbuf.at[pl.ds(off, _GCHUNK)]],
                         rows, sem).wait()
        pltpu.sync_copy(rows, g_hbm.at[pl.ds(base + off, _GCHUNK)])
        return 0
    lax.fori_loop(0, _EPW // _GCHUNK, _chunk, 0)


def _gather_one(tab, src):
    ka = tab.shape[1]
    k = pl.kernel(
        _sc_gather1_body,
        out_type=jax.ShapeDtypeStruct((E, ka), jnp.float32),
        mesh=plsc.VectorSubcoreMesh(**_SC_MESH),
        compiler_params=pltpu.CompilerParams(needs_layout_passes=False),
        scratch_types=[
            pltpu.VMEM((_EPW,), jnp.int32),
            pltpu.VMEM((_GCHUNK, ka), jnp.float32),
            pltpu.SemaphoreType.DMA,
        ],
    )
    return k(tab, src)


def _sc_coef_body(dinv_hbm, src_hbm, ew_hbm, coef_hbm,
                  srcbuf, ewbuf, dinvbuf, coefbuf):
    cid = lax.axis_index("c")
    sid = lax.axis_index("s")
    wid = sid * _NC + cid
    base = wid * _EPW
    pltpu.sync_copy(src_hbm.at[pl.ds(base, _EPW)], srcbuf)
    pltpu.sync_copy(ew_hbm.at[pl.ds(base, _EPW)], ewbuf)
    pltpu.sync_copy(dinv_hbm, dinvbuf)

    def _coef(j, _):
        s16 = srcbuf[pl.ds(j * 16, 16)]
        w16 = ewbuf[pl.ds(j * 16, 16)]
        coefbuf[pl.ds(j * 16, 16)] = plsc.load_gather(dinvbuf, [s16]) * w16
        return 0
    lax.fori_loop(0, _EPW // 16, _coef, 0)
    pltpu.sync_copy(coefbuf, coef_hbm.at[pl.ds(base, _EPW)])


def _coef_pass(dinv, src, ew):
    k = pl.kernel(
        _sc_coef_body,
        out_type=jax.ShapeDtypeStruct((E,), jnp.float32),
        mesh=plsc.VectorSubcoreMesh(**_SC_MESH),
        compiler_params=pltpu.CompilerParams(needs_layout_passes=False),
        scratch_types=[
            pltpu.VMEM((_EPW,), jnp.int32),
            pltpu.VMEM((_EPW,), jnp.float32),
            pltpu.VMEM((N,), jnp.float32),
            pltpu.VMEM((_EPW,), jnp.float32),
        ],
    )
    return k(dinv, src, ew)


def _scatter_rows(dst, msg):
    k = pl.kernel(
        _sc_scatter,
        out_type=jax.ShapeDtypeStruct((_NC, _NPAD, D), jnp.float32),
        mesh=plsc.VectorSubcoreMesh(**_SC_MESH),
        compiler_params=pltpu.CompilerParams(needs_layout_passes=False),
        scratch_types=[
            pltpu.VMEM((_SCHUNK,), jnp.int32),
            pltpu.VMEM((_SCHUNK, D), jnp.float32),
            pltpu.VMEM((_ZROWS, D), jnp.float32),
            pltpu.VMEM_SHARED((_NPAD, D), jnp.float32),
            pltpu.SemaphoreType.DMA,
        ],
    )
    return k(dst, msg)


def _erf(z):
    return jax.lax.erf(z)


def _gelu(x):
    return 0.5 * x * (1.0 + _erf(x * 0.7071067811865476))


def _ln(x, g, b):
    m = jnp.mean(x, axis=-1, keepdims=True)
    v = jnp.mean((x - m) ** 2, axis=-1, keepdims=True)
    return (x - m) * jax.lax.rsqrt(v + 1e-5) * g + b


def _ka_body(x_ref, w_ref, deg_ref, tab_ref, xw1s_ref, dinv_ref):
    xc = jnp.dot(x_ref[:], w_ref[:], preferred_element_type=jnp.float32)
    dinv = jax.lax.rsqrt(deg_ref[:] + 1.0)
    tab_ref[:] = xc[:, D:]
    xw1s_ref[:] = xc[:, :D] * dinv
    dinv_ref[:] = dinv


def _node_matmul_dinv(x, wcat, deg):
    ko = wcat.shape[1]
    return pl.pallas_call(
        _ka_body,
        grid=(N // NB,),
        in_specs=[
            pl.BlockSpec((NB, D), lambda i: (i, 0)),
            pl.BlockSpec((D, ko), lambda i: (0, 0)),
            pl.BlockSpec((NB, 1), lambda i: (i, 0)),
        ],
        out_specs=[
            pl.BlockSpec((NB, ko - D), lambda i: (i, 0)),
            pl.BlockSpec((NB, D), lambda i: (i, 0)),
            pl.BlockSpec((NB, 1), lambda i: (i, 0)),
        ],
        out_shape=[
            jax.ShapeDtypeStruct((N, ko - D), jnp.float32),
            jax.ShapeDtypeStruct((N, D), jnp.float32),
            jax.ShapeDtypeStruct((N, 1), jnp.float32),
        ],
    )(x, wcat, deg)


def _kb_body(xs_ref, xd_ref, we2_ref, wu1c_ref,
             be1_ref, be2_ref, bu1_ref, ec_ref):
    e1 = jax.nn.relu(xs_ref[:, :ED] + xd_ref[:, ED:] + be1_ref[:])
    e = jnp.dot(e1, we2_ref[:], preferred_element_type=jnp.float32) + be2_ref[:]
    ec_ref[:] = jnp.dot(e, wu1c_ref[:], preferred_element_type=jnp.float32) + bu1_ref[:]


def _edge_pass1(gS, gD1, We2, Wu1c, be1, be2, bu1):
    return pl.pallas_call(
        _kb_body,
        grid=(E // EB,),
        in_specs=[
            pl.BlockSpec((EB, 2 * ED), lambda i: (i, 0)),
            pl.BlockSpec((EB, 2 * ED), lambda i: (i, 0)),
            pl.BlockSpec((ED, ED), lambda i: (0, 0)),
            pl.BlockSpec((ED, HD), lambda i: (0, 0)),
            pl.BlockSpec((1, ED), lambda i: (0, 0)),
            pl.BlockSpec((1, ED), lambda i: (0, 0)),
            pl.BlockSpec((1, HD), lambda i: (0, 0)),
        ],
        out_specs=pl.BlockSpec((EB, HD), lambda i: (i, 0)),
        out_shape=jax.ShapeDtypeStruct((E, HD), jnp.float32),
    )(gS, gD1, We2, Wu1c,
      be1.reshape(1, ED), be2.reshape(1, ED), bu1.reshape(1, HD))


def _kc_body(acc_ref, xw_ref, dinv_ref, b_ref, g_ref, bt_ref, w_ref,
             tabc_ref, tabd_ref):
    dinv = dinv_ref[:]
    pre = dinv * (acc_ref[0] + acc_ref[1] + xw_ref[:]) + b_ref[:]
    h = _ln(_gelu(pre), g_ref[:], bt_ref[:])
    hcat = jnp.dot(h, w_ref[:], preferred_element_type=jnp.float32)
    tabc_ref[:] = hcat[:, :2 * D]
    tabd_ref[:] = hcat[:, 2 * D:]


def _node_pass2(acc, xw1s, dinv, b1, g1, bt1, wcat2):
    return pl.pallas_call(
        _kc_body,
        grid=(N // NB,),
        in_specs=[
            pl.BlockSpec((_NC, NB, D), lambda i: (0, i, 0)),
            pl.BlockSpec((NB, D), lambda i: (i, 0)),
            pl.BlockSpec((NB, 1), lambda i: (i, 0)),
            pl.BlockSpec((1, D), lambda i: (0, 0)),
            pl.BlockSpec((1, D), lambda i: (0, 0)),
            pl.BlockSpec((1, D), lambda i: (0, 0)),
            pl.BlockSpec((D, 3 * D), lambda i: (0, 0)),
        ],
        out_specs=[
            pl.BlockSpec((NB, 2 * D), lambda i: (i, 0)),
            pl.BlockSpec((NB, D), lambda i: (i, 0)),
        ],
        out_shape=[
            jax.ShapeDtypeStruct((N, 2 * D), jnp.float32),
            jax.ShapeDtypeStruct((N, D), jnp.float32),
        ],
    )(acc, xw1s, dinv, b1.reshape(1, D), g1.reshape(1, D), bt1.reshape(1, D),
      wcat2)


def _kd_body(has_ref, hbd_ref, ec_ref, wu2_ref, bu2_ref, ge_ref, bte_ref,
             ww_ref, bw_ref, ew_ref):
    u1 = jax.nn.relu(has_ref[:] + hbd_ref[:] + ec_ref[:])
    u = jnp.dot(u1, wu2_ref[:], preferred_element_type=jnp.float32) + bu2_ref[:]
    e2 = _ln(u, ge_ref[:], bte_ref[:])
    logit = jnp.sum(e2 * ww_ref[:], axis=-1, keepdims=True) + bw_ref[:]
    ew_ref[:] = jax.nn.sigmoid(logit)


def _edge_pass2(gC, hbd, ec, Wu2, bu2, ge, bte, Ww, bw):
    return pl.pallas_call(
        _kd_body,
        grid=(E // EB,),
        in_specs=[
            pl.BlockSpec((EB, HD), lambda i: (i, 1)),
            pl.BlockSpec((EB, HD), lambda i: (i, 0)),
            pl.BlockSpec((EB, HD), lambda i: (i, 0)),
            pl.BlockSpec((HD, ED), lambda i: (0, 0)),
            pl.BlockSpec((1, ED), lambda i: (0, 0)),
            pl.BlockSpec((1, ED), lambda i: (0, 0)),
            pl.BlockSpec((1, ED), lambda i: (0, 0)),
            pl.BlockSpec((1, ED), lambda i: (0, 0)),
            pl.BlockSpec((1, 1), lambda i: (0, 0)),
        ],
        out_specs=pl.BlockSpec((EB, 1), lambda i: (i, 0)),
        out_shape=jax.ShapeDtypeStruct((E, 1), jnp.float32),
    )(gC, hbd, ec, Wu2, bu2.reshape(1, ED), ge.reshape(1, ED),
      bte.reshape(1, ED), Ww.reshape(1, ED), bw.reshape(1, 1))


def _kg_body(g2w_ref, m2_ref, msg_ref):
    msg_ref[:] = g2w_ref[:] * m2_ref[:]


def _edge_scale2(gC, m2):
    return pl.pallas_call(
        _kg_body,
        grid=(E // EB,),
        in_specs=[
            pl.BlockSpec((EB, D), lambda i: (i, 0)),
            pl.BlockSpec((EB, 1), lambda i: (i, 0)),
        ],
        out_specs=pl.BlockSpec((EB, D), lambda i: (i, 0)),
        out_shape=jax.ShapeDtypeStruct((E, D), jnp.float32),
    )(gC, m2)


def _kf_body(deg_ref, hw_ref, dinv_ref, hws_ref):
    dinv = jax.lax.rsqrt(deg_ref[:] + 1.0)
    dinv_ref[:] = dinv
    hws_ref[:] = hw_ref[:] * dinv


def _dinv2_scale(deg, tabC):
    return pl.pallas_call(
        _kf_body,
        grid=(N // NB,),
        in_specs=[
            pl.BlockSpec((NB, 1), lambda i: (i, 0)),
            pl.BlockSpec((NB, D), lambda i: (i, 0)),
        ],
        out_specs=[
            pl.BlockSpec((NB, 1), lambda i: (i, 0)),
            pl.BlockSpec((NB, D), lambda i: (i, 0)),
        ],
        out_shape=[
            jax.ShapeDtypeStruct((N, 1), jnp.float32),
            jax.ShapeDtypeStruct((N, D), jnp.float32),
        ],
    )(deg, tabC)


def _ke_body(acc_ref, hws_ref, dinv_ref, b_ref, g_ref, bt_ref, h2_ref):
    pre = dinv_ref[:] * (acc_ref[0] + acc_ref[1] + hws_ref[:]) + b_ref[:]
    h2_ref[:] = _ln(_gelu(pre), g_ref[:], bt_ref[:])


def _node_final(acc, hws, dinv, b2, g2, bt2):
    return pl.pallas_call(
        _ke_body,
        grid=(N // NB,),
        in_specs=[
            pl.BlockSpec((_NC, NB, D), lambda i: (0, i, 0)),
            pl.BlockSpec((NB, D), lambda i: (i, 0)),
            pl.BlockSpec((NB, 1), lambda i: (i, 0)),
            pl.BlockSpec((1, D), lambda i: (0, 0)),
            pl.BlockSpec((1, D), lambda i: (0, 0)),
            pl.BlockSpec((1, D), lambda i: (0, 0)),
        ],
        out_specs=pl.BlockSpec((NB, D), lambda i: (i, 0)),
        out_shape=jax.ShapeDtypeStruct((N, D), jnp.float32),
    )(acc, hws, dinv, b2.reshape(1, D), g2.reshape(1, D), bt2.reshape(1, D))


def kernel(x, edge_index, W1, b1, g1, bt1, W2, b2, g2, bt2, We1, be1, We2,
           be2, Wu1, bu1, Wu2, bu2, ge, bte, Ww, bw):
    src = edge_index[0]
    dst = edge_index[1]

    We1a, We1b = We1[:D], We1[D:]
    Wu1a, Wu1b, Wu1c = Wu1[:D], Wu1[D:2 * D], Wu1[2 * D:]

    ones_e = jnp.ones((E,), jnp.float32)
    deg1 = _histogram(dst, ones_e).sum(axis=0)

    wcat = jnp.concatenate([W1, We1a, We1b], axis=1)
    tabAB, xw1s, dinv1 = _node_matmul_dinv(x, wcat, deg1.reshape(N, 1))

    gX = _gather_one(xw1s, src)
    acc1 = _scatter_rows(dst, gX)

    gS, gD1 = _gather_pass(tabAB, tabAB, src, dst)
    ec = _edge_pass1(gS, gD1, We2, Wu1c, be1, be2, bu1)

    wcat2 = jnp.concatenate([W2, Wu1a, Wu1b], axis=1)
    tabC, tabD = _node_pass2(acc1, xw1s, dinv1, b1, g1, bt1, wcat2)

    gC, gDD = _gather_pass(tabC, tabD, src, dst)

    ew = _edge_pass2(gC, gDD, ec, Wu2, bu2, ge, bte, Ww, bw)
    ewf = ew[:, 0]

    deg2 = _histogram(dst, ewf).sum(axis=0)
    dinv2, h1w2s = _dinv2_scale(deg2.reshape(N, 1), tabC)
    m2 = _coef_pass(dinv2[:, 0], src, ewf)

    msg2 = _edge_scale2(gC, m2.reshape(E, 1))
    acc2 = _scatter_rows(dst, msg2)

    h2 = _node_final(acc2, h1w2s, dinv2, b2, g2, bt2)
    return h2


# speedup vs baseline: 8.6606x; 1.0164x over previous
import functools

import jax
import jax.numpy as jnp
from jax import lax
from jax.experimental import pallas as pl
from jax.experimental.pallas import tpu as pltpu
from jax.experimental.pallas import tpu_sc as plsc

N = 10000
E = 320000
D = 128
ED = 64
HD = 128

NB = 1000
EB = 2560

_NC = 2
_NS = 16
_NW = _NC * _NS
_EPW = E // _NW
_NPS = N // _NS

_SC_MESH = dict(core_axis_name="c", subcore_axis_name="s")


def _sc_hist(idx_hbm, w_hbm, out_hbm, idxbuf, wbuf, acc):
    cid = lax.axis_index("c")
    sid = lax.axis_index("s")
    wid = sid * _NC + cid
    base = wid * _EPW
    pltpu.sync_copy(idx_hbm.at[pl.ds(base, _EPW)], idxbuf)
    pltpu.sync_copy(w_hbm.at[pl.ds(base, _EPW)], wbuf)
    zeros = jnp.zeros((16,), jnp.float32)

    def _zero(i, _):
        acc[pl.ds(i * 16, 16)] = zeros
        return 0
    lax.fori_loop(0, N // 16, _zero, 0)

    def _accum(j, _):
        s16 = idxbuf[pl.ds(j * 16, 16)]
        w16 = wbuf[pl.ds(j * 16, 16)]
        plsc.addupdate_scatter(acc, [s16], w16)
        return 0
    lax.fori_loop(0, _EPW // 16, _accum, 0)
    pltpu.sync_copy(acc, out_hbm.at[wid])


def _histogram(dst, w):
    k = pl.kernel(
        _sc_hist,
        out_type=jax.ShapeDtypeStruct((_NW, N), jnp.float32),
        mesh=plsc.VectorSubcoreMesh(**_SC_MESH),
        compiler_params=pltpu.CompilerParams(needs_layout_passes=False),
        scratch_types=[
            pltpu.VMEM((_EPW,), jnp.int32),
            pltpu.VMEM((_EPW,), jnp.float32),
            pltpu.VMEM((N,), jnp.float32),
        ],
    )
    return k(dst, w)


_SCHUNK = 200
_NPAD = 10240
_NHALF = _NPAD // 2
_ACCR = _NHALF + 8
_EPT = E // _NS
_TROWS2 = _NHALF // _NS
_TROWS = _NPAD // _NS
_ZROWS = 32
_RMI = _SCHUNK // 16
_SEG = 4000


def _zero_spmem_half(zbuf, acc_sh, sid):
    zeros = jnp.zeros((16,), jnp.float32)

    def _zero(i, _):
        zbuf[i // 8, pl.ds((i % 8) * 16, 16)] = zeros
        return 0
    lax.fori_loop(0, _ZROWS * 8, _zero, 0)
    stripe = sid * _TROWS2
    for kk in range(_TROWS2 // _ZROWS):
        pltpu.sync_copy(zbuf, acc_sh.at[pl.ds(stripe + kk * _ZROWS, _ZROWS)])

    @pl.when(sid == 0)
    def _():
        pltpu.sync_copy(zbuf.at[pl.ds(0, 8)], acc_sh.at[pl.ds(_NHALF, 8)])


def _remap_chunk(dstbuf, off, lo, ibuf):
    def _rm(j, _):
        d16 = dstbuf[pl.ds(off + j * 16, 16)]
        v = d16 - lo
        oob = (v < 0) | (v >= _NHALF)
        ibuf[pl.ds(j * 16, 16)] = jnp.where(oob, _NHALF, v)
        return 0
    lax.fori_loop(0, _RMI, _rm, 0)


def _copyout_half(acc_sh, out_hbm, cid, sid):
    stripe = sid * _TROWS2
    for kk in range(_TROWS2 // _ZROWS):
        r0 = stripe + kk * _ZROWS
        pltpu.sync_copy(acc_sh.at[pl.ds(r0, _ZROWS)],
                        out_hbm.at[pl.ds(cid * _NHALF + r0, _ZROWS)])


def _sc_scatter(dst_hbm, msg_hbm, out_hbm, idxbuf, rows, zbuf, acc_sh, sem):
    cid = lax.axis_index("c")
    sid = lax.axis_index("s")
    wid = sid * _NC + cid
    zeros = jnp.zeros((16,), jnp.float32)

    def _zero(i, _):
        zbuf[i // 8, pl.ds((i % 8) * 16, 16)] = zeros
        return 0
    lax.fori_loop(0, _ZROWS * 8, _zero, 0)
    stripe = sid * _TROWS
    for kk in range(_TROWS // _ZROWS):
        pltpu.sync_copy(zbuf, acc_sh.at[pl.ds(stripe + kk * _ZROWS, _ZROWS)])
    plsc.subcore_barrier()

    def _chunk(ch, _):
        base = wid * _EPW + ch * _SCHUNK
        d = pltpu.async_copy(msg_hbm.at[pl.ds(base, _SCHUNK)], rows, sem)
        pltpu.sync_copy(dst_hbm.at[pl.ds(base, _SCHUNK)], idxbuf)
        d.wait()
        pltpu.sync_copy(rows, acc_sh.at[idxbuf], add=True)
        return 0
    lax.fori_loop(0, _EPW // _SCHUNK, _chunk, 0)
    plsc.subcore_barrier()
    for kk in range(_TROWS // _ZROWS):
        r0 = stripe + kk * _ZROWS
        pltpu.sync_copy(acc_sh.at[pl.ds(r0, _ZROWS)],
                        out_hbm.at[cid, pl.ds(r0, _ZROWS)])


_GCHUNK = 200


def _sc_gather_body(gch, tabA_hbm, tabB_hbm, src_hbm, dst_hbm,
                    gA_hbm, gB_hbm,
                    srcbuf, dstbuf, rowsA, rowsB, semA, semB):
    cid = lax.axis_index("c")
    sid = lax.axis_index("s")
    wid = sid * _NC + cid
    base = wid * _EPW
    pltpu.sync_copy(src_hbm.at[pl.ds(base, _EPW)], srcbuf)
    pltpu.sync_copy(dst_hbm.at[pl.ds(base, _EPW)], dstbuf)

    def _chunk(ch, _):
        off = ch * gch
        a = pltpu.async_copy(tabA_hbm.at[srcbuf.at[pl.ds(off, gch)]],
                             rowsA, semA)
        b = pltpu.async_copy(tabB_hbm.at[dstbuf.at[pl.ds(off, gch)]],
                             rowsB, semB)
        a.wait()
        b.wait()
        pltpu.sync_copy(rowsA, gA_hbm.at[pl.ds(base + off, gch)])
        pltpu.sync_copy(rowsB, gB_hbm.at[pl.ds(base + off, gch)])
        return 0
    lax.fori_loop(0, _EPW // gch, _chunk, 0)


def _gather_pass(tabA, tabB, src, dst):
    ka = tabA.shape[1]
    kb = tabB.shape[1]
    gch = 400 if ka + kb <= 2 * D else _GCHUNK
    k = pl.kernel(
        functools.partial(_sc_gather_body, gch),
        out_type=[
            jax.ShapeDtypeStruct((E, ka), jnp.float32),
            jax.ShapeDtypeStruct((E, kb), jnp.float32),
        ],
        mesh=plsc.VectorSubcoreMesh(**_SC_MESH),
        compiler_params=pltpu.CompilerParams(needs_layout_passes=False),
        scratch_types=[
            pltpu.VMEM((_EPW,), jnp.int32),
            pltpu.VMEM((_EPW,), jnp.int32),
            pltpu.VMEM((gch, ka), jnp.float32),
            pltpu.VMEM((gch, kb), jnp.float32),
            pltpu.SemaphoreType.DMA,
            pltpu.SemaphoreType.DMA,
        ],
    )
    return k(tabA, tabB, src, dst)


_G1CH = 400


def _sc_gather1_body(tab_hbm, src_hbm, g_hbm, srcbuf, rows, sem):
    cid = lax.axis_index("c")
    sid = lax.axis_index("s")
    wid = sid * _NC + cid
    base = wid * _EPW
    pltpu.sync_copy(src_hbm.at[pl.ds(base, _EPW)], srcbuf)

    def _chunk(ch, _):
        off = ch * _G1CH
        pltpu.async_copy(tab_hbm.at[srcbuf.at[pl.ds(off, _G1CH)]],
                         rows, sem).wait()
        pltpu.sync_copy(rows, g_hbm.at[pl.ds(base + off, _G1CH)])
        return 0
    lax.fori_loop(0, _EPW // _G1CH, _chunk, 0)


def _gather_one(tab, src):
    ka = tab.shape[1]
    k = pl.kernel(
        _sc_gather1_body,
        out_type=jax.ShapeDtypeStruct((E, ka), jnp.float32),
        mesh=plsc.VectorSubcoreMesh(**_SC_MESH),
        compiler_params=pltpu.CompilerParams(needs_layout_passes=False),
        scratch_types=[
            pltpu.VMEM((_EPW,), jnp.int32),
            pltpu.VMEM((_G1CH, ka), jnp.float32),
            pltpu.SemaphoreType.DMA,
        ],
    )
    return k(tab, src)


def _sc_coef_body(dinv_hbm, src_hbm, ew_hbm, coef_hbm,
                  srcbuf, ewbuf, dinvbuf, coefbuf):
    cid = lax.axis_index("c")
    sid = lax.axis_index("s")
    wid = sid * _NC + cid
    base = wid * _EPW
    pltpu.sync_copy(src_hbm.at[pl.ds(base, _EPW)], srcbuf)
    pltpu.sync_copy(ew_hbm.at[pl.ds(base, _EPW)], ewbuf)
    pltpu.sync_copy(dinv_hbm, dinvbuf)

    def _coef(j, _):
        s16 = srcbuf[pl.ds(j * 16, 16)]
        w16 = ewbuf[pl.ds(j * 16, 16)]
        coefbuf[pl.ds(j * 16, 16)] = plsc.load_gather(dinvbuf, [s16]) * w16
        return 0
    lax.fori_loop(0, _EPW // 16, _coef, 0)
    pltpu.sync_copy(coefbuf, coef_hbm.at[pl.ds(base, _EPW)])


def _coef_pass(dinv, src, ew):
    k = pl.kernel(
        _sc_coef_body,
        out_type=jax.ShapeDtypeStruct((E,), jnp.float32),
        mesh=plsc.VectorSubcoreMesh(**_SC_MESH),
        compiler_params=pltpu.CompilerParams(needs_layout_passes=False),
        scratch_types=[
            pltpu.VMEM((_EPW,), jnp.int32),
            pltpu.VMEM((_EPW,), jnp.float32),
            pltpu.VMEM((N,), jnp.float32),
            pltpu.VMEM((_EPW,), jnp.float32),
        ],
    )
    return k(dinv, src, ew)


def _scatter_rows(dst, msg):
    k = pl.kernel(
        _sc_scatter,
        out_type=jax.ShapeDtypeStruct((_NC, _NPAD, D), jnp.float32),
        mesh=plsc.VectorSubcoreMesh(**_SC_MESH),
        compiler_params=pltpu.CompilerParams(needs_layout_passes=False),
        scratch_types=[
            pltpu.VMEM((_SCHUNK,), jnp.int32),
            pltpu.VMEM((_SCHUNK, D), jnp.float32),
            pltpu.VMEM((_ZROWS, D), jnp.float32),
            pltpu.VMEM_SHARED((_NPAD, D), jnp.float32),
            pltpu.SemaphoreType.DMA,
        ],
    )
    return k(dst, msg)


def _erf(z):
    return jax.lax.erf(z)


def _gelu(x):
    return 0.5 * x * (1.0 + _erf(x * 0.7071067811865476))


def _ln(x, g, b):
    m = jnp.mean(x, axis=-1, keepdims=True)
    v = jnp.mean((x - m) ** 2, axis=-1, keepdims=True)
    return (x - m) * jax.lax.rsqrt(v + 1e-5) * g + b


def _ka_body(x_ref, w_ref, deg_ref, tab_ref, xw1s_ref, dinv_ref):
    xc = jnp.dot(x_ref[:], w_ref[:], preferred_element_type=jnp.float32)
    dinv = jax.lax.rsqrt(deg_ref[:] + 1.0)
    tab_ref[:] = xc[:, D:]
    xw1s_ref[:] = xc[:, :D] * dinv
    dinv_ref[:] = dinv


def _node_matmul_dinv(x, wcat, deg):
    ko = wcat.shape[1]
    return pl.pallas_call(
        _ka_body,
        grid=(N // NB,),
        in_specs=[
            pl.BlockSpec((NB, D), lambda i: (i, 0)),
            pl.BlockSpec((D, ko), lambda i: (0, 0)),
            pl.BlockSpec((NB, 1), lambda i: (i, 0)),
        ],
        out_specs=[
            pl.BlockSpec((NB, ko - D), lambda i: (i, 0)),
            pl.BlockSpec((NB, D), lambda i: (i, 0)),
            pl.BlockSpec((NB, 1), lambda i: (i, 0)),
        ],
        out_shape=[
            jax.ShapeDtypeStruct((N, ko - D), jnp.float32),
            jax.ShapeDtypeStruct((N, D), jnp.float32),
            jax.ShapeDtypeStruct((N, 1), jnp.float32),
        ],
    )(x, wcat, deg)


def _kb_body(xs_ref, xd_ref, we2_ref, wu1c_ref,
             be1_ref, be2_ref, bu1_ref, ec_ref):
    e1 = jax.nn.relu(xs_ref[:, :ED] + xd_ref[:, ED:] + be1_ref[:])
    e = jnp.dot(e1, we2_ref[:], preferred_element_type=jnp.float32) + be2_ref[:]
    ec_ref[:] = jnp.dot(e, wu1c_ref[:], preferred_element_type=jnp.float32) + bu1_ref[:]


def _edge_pass1(gS, gD1, We2, Wu1c, be1, be2, bu1):
    return pl.pallas_call(
        _kb_body,
        grid=(E // EB,),
        in_specs=[
            pl.BlockSpec((EB, 2 * ED), lambda i: (i, 0)),
            pl.BlockSpec((EB, 2 * ED), lambda i: (i, 0)),
            pl.BlockSpec((ED, ED), lambda i: (0, 0)),
            pl.BlockSpec((ED, HD), lambda i: (0, 0)),
            pl.BlockSpec((1, ED), lambda i: (0, 0)),
            pl.BlockSpec((1, ED), lambda i: (0, 0)),
            pl.BlockSpec((1, HD), lambda i: (0, 0)),
        ],
        out_specs=pl.BlockSpec((EB, HD), lambda i: (i, 0)),
        out_shape=jax.ShapeDtypeStruct((E, HD), jnp.float32),
    )(gS, gD1, We2, Wu1c,
      be1.reshape(1, ED), be2.reshape(1, ED), bu1.reshape(1, HD))


def _kc_body(acc_ref, xw_ref, dinv_ref, b_ref, g_ref, bt_ref, w_ref,
             tabc_ref, tabd_ref):
    dinv = dinv_ref[:]
    pre = dinv * (acc_ref[0] + acc_ref[1] + xw_ref[:]) + b_ref[:]
    h = _ln(_gelu(pre), g_ref[:], bt_ref[:])
    hcat = jnp.dot(h, w_ref[:], preferred_element_type=jnp.float32)
    tabc_ref[:] = hcat[:, :2 * D]
    tabd_ref[:] = hcat[:, 2 * D:]


def _node_pass2(acc, xw1s, dinv, b1, g1, bt1, wcat2):
    return pl.pallas_call(
        _kc_body,
        grid=(N // NB,),
        in_specs=[
            pl.BlockSpec((_NC, NB, D), lambda i: (0, i, 0)),
            pl.BlockSpec((NB, D), lambda i: (i, 0)),
            pl.BlockSpec((NB, 1), lambda i: (i, 0)),
            pl.BlockSpec((1, D), lambda i: (0, 0)),
            pl.BlockSpec((1, D), lambda i: (0, 0)),
            pl.BlockSpec((1, D), lambda i: (0, 0)),
            pl.BlockSpec((D, 3 * D), lambda i: (0, 0)),
        ],
        out_specs=[
            pl.BlockSpec((NB, 2 * D), lambda i: (i, 0)),
            pl.BlockSpec((NB, D), lambda i: (i, 0)),
        ],
        out_shape=[
            jax.ShapeDtypeStruct((N, 2 * D), jnp.float32),
            jax.ShapeDtypeStruct((N, D), jnp.float32),
        ],
    )(acc, xw1s, dinv, b1.reshape(1, D), g1.reshape(1, D), bt1.reshape(1, D),
      wcat2)


def _kd_body(has_ref, hbd_ref, ec_ref, wu2_ref, bu2_ref, ge_ref, bte_ref,
             ww_ref, bw_ref, ew_ref):
    u1 = jax.nn.relu(has_ref[:] + hbd_ref[:] + ec_ref[:])
    u = jnp.dot(u1, wu2_ref[:], preferred_element_type=jnp.float32) + bu2_ref[:]
    e2 = _ln(u, ge_ref[:], bte_ref[:])
    logit = jnp.sum(e2 * ww_ref[:], axis=-1, keepdims=True) + bw_ref[:]
    ew_ref[:] = jax.nn.sigmoid(logit)


def _edge_pass2(gC, hbd, ec, Wu2, bu2, ge, bte, Ww, bw):
    return pl.pallas_call(
        _kd_body,
        grid=(E // EB,),
        in_specs=[
            pl.BlockSpec((EB, HD), lambda i: (i, 1)),
            pl.BlockSpec((EB, HD), lambda i: (i, 0)),
            pl.BlockSpec((EB, HD), lambda i: (i, 0)),
            pl.BlockSpec((HD, ED), lambda i: (0, 0)),
            pl.BlockSpec((1, ED), lambda i: (0, 0)),
            pl.BlockSpec((1, ED), lambda i: (0, 0)),
            pl.BlockSpec((1, ED), lambda i: (0, 0)),
            pl.BlockSpec((1, ED), lambda i: (0, 0)),
            pl.BlockSpec((1, 1), lambda i: (0, 0)),
        ],
        out_specs=pl.BlockSpec((EB, 1), lambda i: (i, 0)),
        out_shape=jax.ShapeDtypeStruct((E, 1), jnp.float32),
    )(gC, hbd, ec, Wu2, bu2.reshape(1, ED), ge.reshape(1, ED),
      bte.reshape(1, ED), Ww.reshape(1, ED), bw.reshape(1, 1))


def _kg_body(g2w_ref, m2_ref, msg_ref):
    msg_ref[:] = g2w_ref[:] * m2_ref[:]


def _edge_scale2(gC, m2):
    return pl.pallas_call(
        _kg_body,
        grid=(E // EB,),
        in_specs=[
            pl.BlockSpec((EB, D), lambda i: (i, 0)),
            pl.BlockSpec((EB, 1), lambda i: (i, 0)),
        ],
        out_specs=pl.BlockSpec((EB, D), lambda i: (i, 0)),
        out_shape=jax.ShapeDtypeStruct((E, D), jnp.float32),
    )(gC, m2)


def _kf_body(deg_ref, hw_ref, dinv_ref, hws_ref):
    dinv = jax.lax.rsqrt(deg_ref[:] + 1.0)
    dinv_ref[:] = dinv
    hws_ref[:] = hw_ref[:] * dinv


def _dinv2_scale(deg, tabC):
    return pl.pallas_call(
        _kf_body,
        grid=(N // NB,),
        in_specs=[
            pl.BlockSpec((NB, 1), lambda i: (i, 0)),
            pl.BlockSpec((NB, D), lambda i: (i, 0)),
        ],
        out_specs=[
            pl.BlockSpec((NB, 1), lambda i: (i, 0)),
            pl.BlockSpec((NB, D), lambda i: (i, 0)),
        ],
        out_shape=[
            jax.ShapeDtypeStruct((N, 1), jnp.float32),
            jax.ShapeDtypeStruct((N, D), jnp.float32),
        ],
    )(deg, tabC)


def _ke_body(acc_ref, hws_ref, dinv_ref, b_ref, g_ref, bt_ref, h2_ref):
    pre = dinv_ref[:] * (acc_ref[0] + acc_ref[1] + hws_ref[:]) + b_ref[:]
    h2_ref[:] = _ln(_gelu(pre), g_ref[:], bt_ref[:])


def _node_final(acc, hws, dinv, b2, g2, bt2):
    return pl.pallas_call(
        _ke_body,
        grid=(N // NB,),
        in_specs=[
            pl.BlockSpec((_NC, NB, D), lambda i: (0, i, 0)),
            pl.BlockSpec((NB, D), lambda i: (i, 0)),
            pl.BlockSpec((NB, 1), lambda i: (i, 0)),
            pl.BlockSpec((1, D), lambda i: (0, 0)),
            pl.BlockSpec((1, D), lambda i: (0, 0)),
            pl.BlockSpec((1, D), lambda i: (0, 0)),
        ],
        out_specs=pl.BlockSpec((NB, D), lambda i: (i, 0)),
        out_shape=jax.ShapeDtypeStruct((N, D), jnp.float32),
    )(acc, hws, dinv, b2.reshape(1, D), g2.reshape(1, D), bt2.reshape(1, D))


def kernel(x, edge_index, W1, b1, g1, bt1, W2, b2, g2, bt2, We1, be1, We2,
           be2, Wu1, bu1, Wu2, bu2, ge, bte, Ww, bw):
    src = edge_index[0]
    dst = edge_index[1]

    We1a, We1b = We1[:D], We1[D:]
    Wu1a, Wu1b, Wu1c = Wu1[:D], Wu1[D:2 * D], Wu1[2 * D:]

    ones_e = jnp.ones((E,), jnp.float32)
    deg1 = _histogram(dst, ones_e).sum(axis=0)

    wcat = jnp.concatenate([W1, We1a, We1b], axis=1)
    tabAB, xw1s, dinv1 = _node_matmul_dinv(x, wcat, deg1.reshape(N, 1))

    gX = _gather_one(xw1s, src)
    acc1 = _scatter_rows(dst, gX)

    gS, gD1 = _gather_pass(tabAB, tabAB, src, dst)
    ec = _edge_pass1(gS, gD1, We2, Wu1c, be1, be2, bu1)

    wcat2 = jnp.concatenate([W2, Wu1a, Wu1b], axis=1)
    tabC, tabD = _node_pass2(acc1, xw1s, dinv1, b1, g1, bt1, wcat2)

    gC, gDD = _gather_pass(tabC, tabD, src, dst)

    ew = _edge_pass2(gC, gDD, ec, Wu2, bu2, ge, bte, Ww, bw)
    ewf = ew[:, 0]

    deg2 = _histogram(dst, ewf).sum(axis=0)
    dinv2, h1w2s = _dinv2_scale(deg2.reshape(N, 1), tabC)
    m2 = _coef_pass(dinv2[:, 0], src, ewf)

    msg2 = _edge_scale2(gC, m2.reshape(E, 1))
    acc2 = _scatter_rows(dst, msg2)

    h2 = _node_final(acc2, h1w2s, dinv2, b2, g2, bt2)
    return h2
